# baseline (speedup 1.0000x reference)
import functools

import jax
import jax.numpy as jnp
from jax import lax
from jax.experimental import pallas as pl
from jax.experimental.pallas import tpu as pltpu
from jax.experimental.pallas import tpu_sc as plsc

N = 50000
E = 800000
H = 32
NC2 = 2
NS = 16
NW = NC2 * NS
NACC = 50048
NPC = NACC // NS
ZR = 136
NCNT = 51200
BN = 2000
BE = 8000

_f32 = jnp.float32


@functools.lru_cache(maxsize=None)
def _get_mesh():
    return plsc.VectorSubcoreMesh(core_axis_name="c", subcore_axis_name="s",
                                  num_cores=NC2, num_subcores=NS)


def _zero_vmem_2d(ref, rows):
    def zrow(j, c):
        ref[j, pl.ds(0, 16)] = jnp.zeros((16,), _f32)
        ref[j, pl.ds(16, 16)] = jnp.zeros((16,), _f32)
        return c
    lax.fori_loop(0, rows, zrow, 0)


def _fill_vmem_1d(ref, n, val):
    def z16(i, c):
        ref[pl.ds(i * 16, 16)] = jnp.full((16,), val, _f32)
        return c
    lax.fori_loop(0, n // 16, z16, 0)


def _zero_acc(acc, zrow, K, s):
    q, r = NPC // K, NPC % K

    def zcp(j, c):
        pltpu.sync_copy(zrow, acc.at[pl.ds(s * NPC + j * K, K)])
        return c
    lax.fori_loop(0, q, zcp, 0)
    if r:
        pltpu.sync_copy(zrow.at[pl.ds(0, r)], acc.at[pl.ds(s * NPC + q * K, r)])


def _pipelined(T, issue_idx, wait_idx, issue_gather, wait_gather, consume):
    issue_idx(0, 0)
    wait_idx(0)
    issue_gather(0, 0)
    issue_idx(1, 1)

    def pair(jj, cc):
        j0 = 2 * jj
        j1 = j0 + 1
        wait_gather(0)

        @pl.when(j1 < T)
        def _():
            wait_idx(1)
            issue_gather(1, j1)
        consume(0, j0)

        @pl.when(j0 + 2 < T)
        def _():
            issue_idx(0, j0 + 2)

        @pl.when(j1 < T)
        def _():
            wait_gather(1)

            @pl.when(j1 + 1 < T)
            def _():
                wait_idx(0)
                issue_gather(0, j1 + 1)
            consume(1, j1)

            @pl.when(j1 + 2 < T)
            def _():
                issue_idx(1, j1 + 2)
        return cc
    lax.fori_loop(0, (T + 1) // 2, pair, 0)


_K0 = 1600
_NCH0 = E // _K0


def _sc_deg_gate_body(src, dst, cdp, cdm, counts_out, gate_out,
                      sidx0, sidx1, didx0, didx1, av0, av1, bv0, bv1,
                      onesv, zb1, acc1,
                      sS0, sS1, sD0, sD1, sG0, sG1, sE0, sE1):
    c = lax.axis_index("c")
    s = lax.axis_index("s")
    w = s * NC2 + c
    sidx = (sidx0, sidx1)
    didx = (didx0, didx1)
    av = (av0, av1)
    bv = (bv0, bv1)
    sS = (sS0, sS1)
    sD = (sD0, sD1)
    sG = (sG0, sG1)
    sE = (sE0, sE1)
    _fill_vmem_1d(zb1, _K0, 0.0)
    _fill_vmem_1d(onesv, _K0, 1.0)

    def zcp(j, cc):
        pltpu.sync_copy(zb1, acc1.at[pl.ds(s * 3200 + j * _K0, _K0)])
        return cc
    lax.fori_loop(0, 3200 // _K0, zcp, 0)
    plsc.subcore_barrier()
    T = (_NCH0 - w + NW - 1) // NW

    def base(j):
        return (w + NW * j) * _K0

    def issue_idx(b, j):
        pltpu.async_copy(src.at[pl.ds(base(j), _K0)], sidx[b], sS[b])
        pltpu.async_copy(dst.at[pl.ds(base(j), _K0)], didx[b], sD[b])

    def wait_idx(b):
        pltpu.make_async_copy(src.at[pl.ds(0, _K0)], sidx[b], sS[b]).wait()
        pltpu.make_async_copy(dst.at[pl.ds(0, _K0)], didx[b], sD[b]).wait()

    def issue_gather(b, j):
        pltpu.async_copy(cdp.at[sidx[b]], av[b], sG[b])
        pltpu.async_copy(cdm.at[didx[b]], bv[b], sE[b])

    def wait_gather(b):
        pltpu.make_async_copy(cdp.at[sidx[b]], av[b], sG[b]).wait()
        pltpu.make_async_copy(cdm.at[didx[b]], bv[b], sE[b]).wait()

    def consume(b, j):
        pltpu.sync_copy(onesv, acc1.at[didx[b]], add=True)

        def gfn(i, c2):
            z = av[b][pl.ds(i * 16, 16)] - bv[b][pl.ds(i * 16, 16)]
            av[b][pl.ds(i * 16, 16)] = 1.0 / (1.0 + jnp.exp(-z))
            return c2
        lax.fori_loop(0, _K0 // 16, gfn, 0)
        pltpu.sync_copy(av[b], gate_out.at[pl.ds(base(j), _K0)])

    _pipelined(T, issue_idx, wait_idx, issue_gather, wait_gather, consume)
    plsc.subcore_barrier()
    pltpu.sync_copy(acc1.at[pl.ds(s * 3200, 3200)],
                    counts_out.at[c, pl.ds(s * 3200, 3200)])


@functools.lru_cache(maxsize=None)
def _sc_deg_gate():
    return pl.kernel(
        _sc_deg_gate_body,
        out_type=(jax.ShapeDtypeStruct((NC2, NCNT), _f32),
                  jax.ShapeDtypeStruct((E,), _f32)),
        mesh=_get_mesh(),
        compiler_params=pltpu.CompilerParams(use_tc_tiling_on_sc=False),
        scratch_types=(
            [pltpu.VMEM((_K0,), jnp.int32)] * 4
            + [pltpu.VMEM((_K0,), _f32)] * 6
            + [pltpu.VMEM_SHARED((NCNT,), _f32)]
            + [pltpu.SemaphoreType.DMA] * 8
        ),
    )


@functools.lru_cache(maxsize=None)
def _make_scatter(kind, K):
    nch = E // K

    def body(src, dst, table, *rest):
        if kind == "plain":
            (out, sidx0, sidx1, didx0, didx1, rows0, rows1, acc,
             sS0, sS1, sD0, sD1, sG0, sG1) = rest
            sE = ex = None
        elif kind == "erow":
            (erow, out, sidx0, sidx1, didx0, didx1, rows0, rows1,
             ex0, ex1, acc, sS0, sS1, sD0, sD1, sG0, sG1, sE0, sE1) = rest
            ex = (ex0, ex1)
            sE = (sE0, sE1)
        else:
            (gateh, out, sidx0, sidx1, didx0, didx1, rows0, rows1,
             ex0, ex1, acc, sS0, sS1, sD0, sD1, sG0, sG1, sE0, sE1) = rest
            ex = (ex0, ex1)
            sE = (sE0, sE1)
        c = lax.axis_index("c")
        s = lax.axis_index("s")
        w = s * NC2 + c
        sidx = (sidx0, sidx1)
        didx = (didx0, didx1)
        rows = (rows0, rows1)
        sS = (sS0, sS1)
        sD = (sD0, sD1)
        sG = (sG0, sG1)
        _zero_vmem_2d(rows0, K)
        _zero_acc(acc, rows0, K, s)
        plsc.subcore_barrier()
        T = (nch - w + NW - 1) // NW

        def base(j):
            return (w + NW * j) * K

        def issue_idx(b, j):
            pltpu.async_copy(src.at[pl.ds(base(j), K)], sidx[b], sS[b])
            pltpu.async_copy(dst.at[pl.ds(base(j), K)], didx[b], sD[b])

        def wait_idx(b):
            pltpu.make_async_copy(src.at[pl.ds(0, K)], sidx[b], sS[b]).wait()
            pltpu.make_async_copy(dst.at[pl.ds(0, K)], didx[b], sD[b]).wait()

        def issue_gather(b, j):
            pltpu.async_copy(table.at[sidx[b]], rows[b], sG[b])
            if kind == "erow":
                pltpu.async_copy(erow.at[pl.ds(base(j), K)], ex[b], sE[b])
            elif kind == "gate":
                pltpu.async_copy(gateh.at[pl.ds(base(j), K)], ex[b], sE[b])

        def wait_gather(b):
            pltpu.make_async_copy(table.at[sidx[b]], rows[b], sG[b]).wait()
            if kind == "erow":
                pltpu.make_async_copy(erow.at[pl.ds(0, K)], ex[b], sE[b]).wait()
            elif kind == "gate":
                pltpu.make_async_copy(gateh.at[pl.ds(0, K)], ex[b], sE[b]).wait()

        def consume(b, j):
            if kind == "gate":
                def rowfn(i, c2):
                    gv = ex[b][pl.ds(i * 16, 16)]
                    for l in range(16):
                        k = i * 16 + l
                        g = gv[l]
                        rows[b][k, pl.ds(0, 16)] = rows[b][k, pl.ds(0, 16)] * g
                        rows[b][k, pl.ds(16, 16)] = rows[b][k, pl.ds(16, 16)] * g
                    return c2
                lax.fori_loop(0, K // 16, rowfn, 0)
            pltpu.sync_copy(rows[b], acc.at[didx[b]], add=True)
            if kind == "erow":
                pltpu.sync_copy(ex[b], acc.at[didx[b]], add=True)

        _pipelined(T, issue_idx, wait_idx, issue_gather, wait_gather, consume)
        plsc.subcore_barrier()
        pltpu.sync_copy(acc.at[pl.ds(s * NPC, NPC)],
                        out.at[c, pl.ds(s * NPC, NPC)])

    scratch = [pltpu.VMEM((K,), jnp.int32)] * 4 + [pltpu.VMEM((K, H), _f32)] * 2
    nsem = 6
    if kind == "erow":
        scratch += [pltpu.VMEM((K, H), _f32)] * 2
        nsem = 8
    elif kind == "gate":
        scratch += [pltpu.VMEM((K,), _f32)] * 2
        nsem = 8
    scratch += [pltpu.VMEM_SHARED((NACC, H), _f32)]
    scratch += [pltpu.SemaphoreType.DMA] * nsem
    return pl.kernel(
        body,
        out_type=jax.ShapeDtypeStruct((NC2, NACC, H), _f32),
        mesh=_get_mesh(),
        compiler_params=pltpu.CompilerParams(use_tc_tiling_on_sc=False),
        scratch_types=scratch,
    )


def _sc_scatter_plain(*args):
    return _make_scatter("plain", 400)(*args)


def _sc_scatter_gate(*args):
    return _make_scatter("gate", 400)(*args)


_K3 = 128
_NCH3 = E // _K3


def _sc_escatter_body(src, dst, ta, tb, ch, out,
                      sidx0, sidx1, didx0, didx1, ra0, ra1, rb0, rb1,
                      cc0, cc1, acc,
                      sS0, sS1, sD0, sD1, sG0, sG1, sE0, sE1, sC0, sC1):
    c = lax.axis_index("c")
    s = lax.axis_index("s")
    w = s * NC2 + c
    sidx = (sidx0, sidx1)
    didx = (didx0, didx1)
    ra = (ra0, ra1)
    rb = (rb0, rb1)
    cc = (cc0, cc1)
    sS = (sS0, sS1)
    sD = (sD0, sD1)
    sG = (sG0, sG1)
    sE = (sE0, sE1)
    sC = (sC0, sC1)
    _zero_vmem_2d(ra0, _K3)
    _zero_acc(acc, ra0, _K3, s)
    plsc.subcore_barrier()
    T = (_NCH3 - w + NW - 1) // NW

    def base(j):
        return (w + NW * j) * _K3

    def issue_idx(b, j):
        pltpu.async_copy(src.at[pl.ds(base(j), _K3)], sidx[b], sS[b])
        pltpu.async_copy(dst.at[pl.ds(base(j), _K3)], didx[b], sD[b])

    def wait_idx(b):
        pltpu.make_async_copy(src.at[pl.ds(0, _K3)], sidx[b], sS[b]).wait()
        pltpu.make_async_copy(dst.at[pl.ds(0, _K3)], didx[b], sD[b]).wait()

    def issue_gather(b, j):
        pltpu.async_copy(ta.at[sidx[b]], ra[b], sG[b])
        pltpu.async_copy(tb.at[didx[b]], rb[b], sE[b])
        pltpu.async_copy(ch.at[pl.ds(base(j), _K3)], cc[b], sC[b])

    def wait_gather(b):
        pltpu.make_async_copy(ta.at[sidx[b]], ra[b], sG[b]).wait()
        pltpu.make_async_copy(tb.at[didx[b]], rb[b], sE[b]).wait()
        pltpu.make_async_copy(ch.at[pl.ds(0, _K3)], cc[b], sC[b]).wait()

    def consume(b, j):
        def efn(k, c2):
            for hh in (0, 16):
                v = (ra[b][k, pl.ds(hh, 16)] + rb[b][k, pl.ds(hh, 16)]
                     + cc[b][k, pl.ds(hh, 16)])
                ra[b][k, pl.ds(hh, 16)] = jnp.maximum(v, 0.0)
            return c2
        lax.fori_loop(0, _K3, efn, 0)
        pltpu.sync_copy(ra[b], acc.at[didx[b]], add=True)

    _pipelined(T, issue_idx, wait_idx, issue_gather, wait_gather, consume)
    plsc.subcore_barrier()
    pltpu.sync_copy(acc.at[pl.ds(s * NPC, NPC)],
                    out.at[c, pl.ds(s * NPC, NPC)])


@functools.lru_cache(maxsize=None)
def _sc_escatter():
    return pl.kernel(
        _sc_escatter_body,
        out_type=jax.ShapeDtypeStruct((NC2, NACC, H), _f32),
        mesh=_get_mesh(),
        compiler_params=pltpu.CompilerParams(use_tc_tiling_on_sc=False),
        scratch_types=(
            [pltpu.VMEM((_K3,), jnp.int32)] * 4
            + [pltpu.VMEM((_K3, H), _f32)] * 6
            + [pltpu.VMEM_SHARED((NACC, H), _f32)]
            + [pltpu.SemaphoreType.DMA] * 10
        ),
    )


def _rows(d, bn=BN):
    return pl.BlockSpec((bn, d), lambda i: (i, 0))


def _full(a):
    nd = a.ndim
    return pl.BlockSpec(a.shape, lambda i, _nd=nd: (0,) * _nd)


def _agg_spec(bn=BN):
    return pl.BlockSpec((NC2, bn, H), lambda i: (0, i, 0))


def _relu(x):
    return jnp.maximum(x, 0.0)


def _sigm(x):
    return 1.0 / (1.0 + jnp.exp(-x))


def _tc(body, in_specs, out_shapes, out_specs, grid):
    return pl.pallas_call(
        body,
        grid=(grid,),
        in_specs=in_specs,
        out_specs=out_specs,
        out_shape=out_shapes,
    )


def _sds(*shape):
    return jax.ShapeDtypeStruct(shape, _f32)


def kernel(discrete_x, continous_x, edge_index, edge_attr, churn_date, t, params):
    p = params
    src = edge_index[0]
    dst = edge_index[1]
    cd = churn_date[:, 0]
    cdp = cd * p['g1_wg'] + 0.5 * p['g1_bg']
    cdm = cd * p['g1_wg'] - 0.5 * p['g1_bg']

    counts2, gate = _sc_deg_gate()(src, dst, cdp, cdm)
    counts2 = counts2[:, :N, None]

    elW_A = p['el_W'][:H]
    elW_B = p['el_W'][H:2 * H]
    elW_C = p['el_W'][2 * H:]

    def tc1(disc, cont, cnt2, Wd, bd, Wc, bc, g0W, g0b, g1W,
            xd_o, xc_o, h1_o, hn1_o, norm_o, invdeg_o):
        cnt = cnt2[0] + cnt2[1]
        norm = lax.rsqrt(cnt + 1.0)
        invdeg = 1.0 / jnp.maximum(cnt, 1.0)
        norm_o[...] = norm
        invdeg_o[...] = invdeg
        xd = jnp.dot(disc[...], Wd[...], preferred_element_type=_f32, precision=lax.Precision.HIGHEST) + bd[...]
        xd_o[...] = xd
        cont_v = cont[...]
        xcs = [_relu(jnp.dot(cont_v[:, 16 * i:16 * (i + 1)], Wc[...],
                             preferred_element_type=_f32, precision=lax.Precision.HIGHEST) + bc[...])
               for i in range(3)]
        xc = jnp.concatenate(xcs, axis=-1)
        xc_o[...] = xc
        xg = _relu(jnp.dot(jnp.concatenate([xd, xc], axis=-1), g0W[...],
                           preferred_element_type=_f32, precision=lax.Precision.HIGHEST) + g0b[...])
        h1 = jnp.dot(xg, g1W[...], preferred_element_type=_f32, precision=lax.Precision.HIGHEST)
        h1_o[...] = h1
        hn1_o[...] = h1 * norm

    xd, xc, h1, hn1, normv, invdeg = _tc(
        tc1,
        [_rows(16), _rows(48), pl.BlockSpec((NC2, BN, 1), lambda i: (0, i, 0)),
         _full(p['W_d']), _full(p['b_d']), _full(p['W_c']), _full(p['b_c']),
         _full(p['g0_W']), _full(p['g0_b']), _full(p['gcn1_W'])],
        (_sds(N, 10), _sds(N, 24), _sds(N, H), _sds(N, H), _sds(N, 1), _sds(N, 1)),
        (_rows(10), _rows(24), _rows(H), _rows(H), _rows(1), _rows(1)),
        25,
    )(discrete_x, continous_x, counts2, p['W_d'], p['b_d'], p['W_c'], p['b_c'],
      p['g0_W'], p['g0_b'], p['gcn1_W'])

    agg1 = _sc_scatter_plain(src, dst, hn1)

    def tc2(agg, h1r, nr, g2W, b1, xg0_o, h2_o, hn2_o):
        nv = nr[...]
        xg0 = _relu((agg[0] + agg[1]) * nv + h1r[...] * nv * nv + b1[...])
        xg0_o[...] = xg0
        h2 = jnp.dot(xg0, g2W[...], preferred_element_type=_f32, precision=lax.Precision.HIGHEST)
        h2_o[...] = h2
        hn2_o[...] = h2 * nv

    xg0, h2, hn2 = _tc(
        tc2,
        [_agg_spec(), _rows(H), _rows(1), _full(p['gcn2_W']), _full(p['gcn1_b'])],
        (_sds(N, H), _sds(N, H), _sds(N, H)),
        (_rows(H), _rows(H), _rows(H)),
        25,
    )(agg1, h1, normv, p['gcn2_W'], p['gcn1_b'])

    agg2 = _sc_scatter_plain(src, dst, hn2)

    def tc3(agg, h2r, nr, xdr, xcr, xg0r, b2,
            res1W, res1b, res2W, res2b, cr1w, cr1b, cr2w, cr2b, fuW, fub,
            si0W, si0b, eWA, eWB, el1Wx, el1Wr, el1b, c0W, c0b, g1Wm, g1Wr, g1b,
            hci_o, xsi_o, A_o, B_o, hx1_o, xr1b_o, xns_o, hm1_o, xrn1b_o):
        nv = nr[...]
        xg1 = _relu((agg[0] + agg[1]) * nv + h2r[...] * nv * nv + b2[...])
        x = jnp.concatenate([xdr[...], xcr[...], xg0r[...] + xg1], axis=-1)
        h1r = _relu(jnp.dot(x, res1W[...], preferred_element_type=_f32, precision=lax.Precision.HIGHEST) + res1b[...]) + x
        x_deep = _relu(jnp.dot(h1r, res2W[...], preferred_element_type=_f32, precision=lax.Precision.HIGHEST) + res2b[...]) + h1r
        xl = x
        s1 = jnp.dot(xl, cr1w[...], preferred_element_type=_f32, precision=lax.Precision.HIGHEST)
        xl = x * s1 + cr1b[...] + xl
        s2 = jnp.dot(xl, cr2w[...], preferred_element_type=_f32, precision=lax.Precision.HIGHEST)
        xl = x * s2 + cr2b[...] + xl
        hci_o[...] = _relu(jnp.dot(x_deep + xl, fuW[...], preferred_element_type=_f32, precision=lax.Precision.HIGHEST) + fub[...])
        xsi = _relu(jnp.dot(x, si0W[...], preferred_element_type=_f32, precision=lax.Precision.HIGHEST) + si0b[...])
        xsi_o[...] = xsi
        A_o[...] = jnp.dot(xsi, eWA[...], preferred_element_type=_f32, precision=lax.Precision.HIGHEST)
        B_o[...] = jnp.dot(xsi, eWB[...], preferred_element_type=_f32, precision=lax.Precision.HIGHEST)
        hx1_o[...] = jnp.dot(xsi, el1Wx[...], preferred_element_type=_f32, precision=lax.Precision.HIGHEST)
        xr1b_o[...] = jnp.dot(xsi, el1Wr[...], preferred_element_type=_f32, precision=lax.Precision.HIGHEST) + el1b[...]
        xns = _relu(jnp.dot(x, c0W[...], preferred_element_type=_f32, precision=lax.Precision.HIGHEST) + c0b[...])
        xns_o[...] = xns
        hm1_o[...] = jnp.dot(xns, g1Wm[...], preferred_element_type=_f32, precision=lax.Precision.HIGHEST)
        xrn1b_o[...] = jnp.dot(xns, g1Wr[...], preferred_element_type=_f32, precision=lax.Precision.HIGHEST) + g1b[...]

    cr1w = p['cr1_w'][:, None]
    cr2w = p['cr2_w'][:, None]
    h_ci, x_si, A, B, hx1, xr1b, x_ns, hm1, xrn1b = _tc(
        tc3,
        [_agg_spec(), _rows(H), _rows(1), _rows(10), _rows(24), _rows(H),
         _full(p['gcn2_b']),
         _full(p['res1_W']), _full(p['res1_b']), _full(p['res2_W']), _full(p['res2_b']),
         _full(cr1w), _full(p['cr1_b']), _full(cr2w), _full(p['cr2_b']),
         _full(p['fu_W']), _full(p['fu_b']),
         _full(p['si0_W']), _full(p['si0_b']), _full(elW_A), _full(elW_B),
         _full(p['el1_Wx']), _full(p['el1_Wr']), _full(p['el1_b']),
         _full(p['c0_W']), _full(p['c0_b']),
         _full(p['g1_Wm']), _full(p['g1_Wr']), _full(p['g1_b'])],
        tuple(_sds(N, H) for _ in range(9)),
        tuple(_rows(H) for _ in range(9)),
        25,
    )(agg2, h2, normv, xd, xc, xg0, p['gcn2_b'],
      p['res1_W'], p['res1_b'], p['res2_W'], p['res2_b'],
      cr1w, p['cr1_b'], cr2w, p['cr2_b'], p['fu_W'], p['fu_b'],
      p['si0_W'], p['si0_b'], elW_A, elW_B,
      p['el1_Wx'], p['el1_Wr'], p['el1_b'], p['c0_W'], p['c0_b'],
      p['g1_Wm'], p['g1_Wr'], p['g1_b'])

    def tce1(ea, W, b, C_o):
        C_o[...] = jnp.dot(ea[...], W[...], preferred_element_type=_f32, precision=lax.Precision.HIGHEST) + b[...]

    C = _tc(tce1, [_rows(16, BE), _full(elW_C), _full(p['el_b'])],
            _sds(E, H), _rows(H, BE), 100)(edge_attr, elW_C, p['el_b'])

    S_e = _sc_escatter()(src, dst, A, B, C)

    sagg1 = _sc_scatter_plain(src, dst, hx1)

    def tc4(agg, se, idg, xr1br, We1, el2Wx, el2Wr, el2b, xsi0_o, hx2_o, xr2b_o):
        eterm = jnp.dot(se[0] + se[1], We1[...], preferred_element_type=_f32, precision=lax.Precision.HIGHEST)
        xsi0 = _relu((agg[0] + agg[1] + eterm) * idg[...] + xr1br[...])
        xsi0_o[...] = xsi0
        hx2_o[...] = jnp.dot(xsi0, el2Wx[...], preferred_element_type=_f32, precision=lax.Precision.HIGHEST)
        xr2b_o[...] = jnp.dot(xsi0, el2Wr[...], preferred_element_type=_f32, precision=lax.Precision.HIGHEST) + el2b[...]

    x_si0, hx2, xr2b = _tc(
        tc4,
        [_agg_spec(), _agg_spec(), _rows(1), _rows(H), _full(p['el1_We']),
         _full(p['el2_Wx']), _full(p['el2_Wr']), _full(p['el2_b'])],
        (_sds(N, H), _sds(N, H), _sds(N, H)), (_rows(H), _rows(H), _rows(H)), 25,
    )(sagg1, S_e, invdeg, xr1b, p['el1_We'],
      p['el2_Wx'], p['el2_Wr'], p['el2_b'])

    sagg2 = _sc_scatter_plain(src, dst, hx2)

    def tc5(agg, se, idg, xr2br, xsi0r, We2, TW, Tb, hsi_o, predT_o):
        eterm = jnp.dot(se[0] + se[1], We2[...], preferred_element_type=_f32, precision=lax.Precision.HIGHEST)
        xsi1 = _relu((agg[0] + agg[1] + eterm) * idg[...] + xr2br[...])
        hsi = xsi0r[...] + xsi1
        hsi_o[...] = hsi
        predT_o[...] = _sigm(jnp.dot(hsi, TW[...], preferred_element_type=_f32, precision=lax.Precision.HIGHEST) + Tb[...])

    h_si, pred_T = _tc(
        tc5,
        [_agg_spec(), _agg_spec(), _rows(1), _rows(H), _rows(H),
         _full(p['el2_We']), _full(p['T_W']), _full(p['T_b'])],
        (_sds(N, H), _sds(N, 1)), (_rows(H), _rows(1)), 25,
    )(sagg2, S_e, invdeg, xr2b, x_si0, p['el2_We'], p['T_W'], p['T_b'])

    gagg1 = _sc_scatter_gate(src, dst, hm1, gate)

    def tc6(agg, idg, xrn1br, g1Wm, g1Wr, g1b, xns0_o, hm2_o, xrn2b_o):
        xns0 = _relu((agg[0] + agg[1]) * idg[...] + xrn1br[...])
        xns0_o[...] = xns0
        hm2_o[...] = jnp.dot(xns0, g1Wm[...], preferred_element_type=_f32, precision=lax.Precision.HIGHEST)
        xrn2b_o[...] = jnp.dot(xns0, g1Wr[...], preferred_element_type=_f32, precision=lax.Precision.HIGHEST) + g1b[...]

    x_ns0, hm2, xrn2b = _tc(
        tc6,
        [_agg_spec(), _rows(1), _rows(H),
         _full(p['g1_Wm']), _full(p['g1_Wr']), _full(p['g1_b'])],
        (_sds(N, H), _sds(N, H), _sds(N, H)), (_rows(H), _rows(H), _rows(H)), 25,
    )(gagg1, invdeg, xrn1b, p['g1_Wm'], p['g1_Wr'], p['g1_b'])

    gagg2 = _sc_scatter_gate(src, dst, hm2, gate)

    def tc7(agg, idg, xrn2br, xns0r, hcir, hsir, tr,
            a0W, a0b, a1W, a1b, y0hW, y0hb, y0oW, y0ob, y1hW, y1hb, y1oW, y1ob,
            py_o, pycf_o, py0_o, py1_o):
        xns1 = _relu((agg[0] + agg[1]) * idg[...] + xrn2br[...])
        hns = xns0r[...] + xns1
        hci = hcir[...]
        hsi = hsir[...]
        h = jnp.concatenate([hci, hsi, hns], axis=-1)
        a0 = jax.nn.softmax(jnp.dot(h, a0W[...], preferred_element_type=_f32, precision=lax.Precision.HIGHEST) + a0b[...], axis=-1)
        py0 = a0[:, :H] * hci + a0[:, H:2 * H] * hsi + a0[:, 2 * H:] * hns
        a1 = jax.nn.softmax(jnp.dot(h, a1W[...], preferred_element_type=_f32, precision=lax.Precision.HIGHEST) + a1b[...], axis=-1)
        py1 = a1[:, :H] * hci + a1[:, H:2 * H] * hsi + a1[:, 2 * H:] * hns
        py0 = _sigm(jnp.dot(_relu(jnp.dot(py0, y0hW[...], preferred_element_type=_f32, precision=lax.Precision.HIGHEST) + y0hb[...]),
                            y0oW[...], preferred_element_type=_f32, precision=lax.Precision.HIGHEST) + y0ob[...])
        py1 = _sigm(jnp.dot(_relu(jnp.dot(py1, y1hW[...], preferred_element_type=_f32, precision=lax.Precision.HIGHEST) + y1hb[...]),
                            y1oW[...], preferred_element_type=_f32, precision=lax.Precision.HIGHEST) + y1ob[...])
        tv = tr[...]
        py_o[...] = (1.0 - tv) * py0 + tv * py1
        pycf_o[...] = tv * py0 + (1.0 - tv) * py1
        py0_o[...] = py0
        py1_o[...] = py1

    pred_y, pred_y_cf, pred_y0, pred_y1 = _tc(
        tc7,
        [_agg_spec(), _rows(1), _rows(H), _rows(H), _rows(H), _rows(H), _rows(1),
         _full(p['a0_W']), _full(p['a0_b']), _full(p['a1_W']), _full(p['a1_b']),
         _full(p['y0h_W']), _full(p['y0h_b']), _full(p['y0o_W']), _full(p['y0o_b']),
         _full(p['y1h_W']), _full(p['y1h_b']), _full(p['y1o_W']), _full(p['y1o_b'])],
        (_sds(N, 1), _sds(N, 1), _sds(N, 1), _sds(N, 1)),
        (_rows(1), _rows(1), _rows(1), _rows(1)),
        25,
    )(gagg2, invdeg, xrn2b, x_ns0, h_ci, h_si, t,
      p['a0_W'], p['a0_b'], p['a1_W'], p['a1_b'],
      p['y0h_W'], p['y0h_b'], p['y0o_W'], p['y0o_b'],
      p['y1h_W'], p['y1h_b'], p['y1o_W'], p['y1o_b'])

    return (pred_y, pred_y_cf, pred_y0, pred_y1, pred_T, h_ci, h_si)


# speedup vs baseline: 1.2321x; 1.2321x over previous
import functools

import jax
import jax.numpy as jnp
from jax import lax
from jax.experimental import pallas as pl
from jax.experimental.pallas import tpu as pltpu
from jax.experimental.pallas import tpu_sc as plsc

N = 50000
E = 800000
H = 32
NC2 = 2
NS = 16
NW = NC2 * NS
NACC = 50048
NPC = NACC // NS
ZR = 136
NCNT = 51200
BN = 2000
BE = 8000

_f32 = jnp.float32


@functools.lru_cache(maxsize=None)
def _get_mesh():
    return plsc.VectorSubcoreMesh(core_axis_name="c", subcore_axis_name="s",
                                  num_cores=NC2, num_subcores=NS)


def _zero_vmem_2d(ref, rows):
    def zrow(j, c):
        ref[j, pl.ds(0, 16)] = jnp.zeros((16,), _f32)
        ref[j, pl.ds(16, 16)] = jnp.zeros((16,), _f32)
        return c
    lax.fori_loop(0, rows, zrow, 0)


def _fill_vmem_1d(ref, n, val):
    def z16(i, c):
        ref[pl.ds(i * 16, 16)] = jnp.full((16,), val, _f32)
        return c
    lax.fori_loop(0, n // 16, z16, 0)


def _zero_acc(acc, zrow, K, s):
    q, r = NPC // K, NPC % K

    def zcp(j, c):
        pltpu.sync_copy(zrow, acc.at[pl.ds(s * NPC + j * K, K)])
        return c
    lax.fori_loop(0, q, zcp, 0)
    if r:
        pltpu.sync_copy(zrow.at[pl.ds(0, r)], acc.at[pl.ds(s * NPC + q * K, r)])


def _pipelined(T, issue_idx, wait_idx, issue_gather, wait_gather, consume):
    issue_idx(0, 0)
    wait_idx(0)
    issue_gather(0, 0)
    issue_idx(1, 1)

    def pair(jj, cc):
        j0 = 2 * jj
        j1 = j0 + 1
        wait_gather(0)

        @pl.when(j1 < T)
        def _():
            wait_idx(1)
            issue_gather(1, j1)
        consume(0, j0)

        @pl.when(j0 + 2 < T)
        def _():
            issue_idx(0, j0 + 2)

        @pl.when(j1 < T)
        def _():
            wait_gather(1)

            @pl.when(j1 + 1 < T)
            def _():
                wait_idx(0)
                issue_gather(0, j1 + 1)
            consume(1, j1)

            @pl.when(j1 + 2 < T)
            def _():
                issue_idx(1, j1 + 2)
        return cc
    lax.fori_loop(0, (T + 1) // 2, pair, 0)


_K0 = 1600
_NCH0 = E // _K0


def _sc_deg_gate_body(src, dst, cdp, cdm, counts_out, gate_out,
                      sidx0, sidx1, didx0, didx1, av0, av1, bv0, bv1,
                      onesv, zb1, acc1,
                      sS0, sS1, sD0, sD1, sG0, sG1, sE0, sE1):
    c = lax.axis_index("c")
    s = lax.axis_index("s")
    w = s * NC2 + c
    sidx = (sidx0, sidx1)
    didx = (didx0, didx1)
    av = (av0, av1)
    bv = (bv0, bv1)
    sS = (sS0, sS1)
    sD = (sD0, sD1)
    sG = (sG0, sG1)
    sE = (sE0, sE1)
    _fill_vmem_1d(zb1, _K0, 0.0)
    _fill_vmem_1d(onesv, _K0, 1.0)

    def zcp(j, cc):
        pltpu.sync_copy(zb1, acc1.at[pl.ds(s * 3200 + j * _K0, _K0)])
        return cc
    lax.fori_loop(0, 3200 // _K0, zcp, 0)
    plsc.subcore_barrier()
    T = (_NCH0 - w + NW - 1) // NW

    def base(j):
        return (w + NW * j) * _K0

    def issue_idx(b, j):
        pltpu.async_copy(src.at[pl.ds(base(j), _K0)], sidx[b], sS[b])
        pltpu.async_copy(dst.at[pl.ds(base(j), _K0)], didx[b], sD[b])

    def wait_idx(b):
        pltpu.make_async_copy(src.at[pl.ds(0, _K0)], sidx[b], sS[b]).wait()
        pltpu.make_async_copy(dst.at[pl.ds(0, _K0)], didx[b], sD[b]).wait()

    def issue_gather(b, j):
        pltpu.async_copy(cdp.at[sidx[b]], av[b], sG[b])
        pltpu.async_copy(cdm.at[didx[b]], bv[b], sE[b])

    def wait_gather(b):
        pltpu.make_async_copy(cdp.at[sidx[b]], av[b], sG[b]).wait()
        pltpu.make_async_copy(cdm.at[didx[b]], bv[b], sE[b]).wait()

    def consume(b, j):
        pltpu.sync_copy(onesv, acc1.at[didx[b]], add=True)

        def gfn(i, c2):
            z = av[b][pl.ds(i * 16, 16)] - bv[b][pl.ds(i * 16, 16)]
            av[b][pl.ds(i * 16, 16)] = 1.0 / (1.0 + jnp.exp(-z))
            return c2
        lax.fori_loop(0, _K0 // 16, gfn, 0)
        pltpu.sync_copy(av[b], gate_out.at[pl.ds(base(j), _K0)])

    _pipelined(T, issue_idx, wait_idx, issue_gather, wait_gather, consume)
    plsc.subcore_barrier()
    pltpu.sync_copy(acc1.at[pl.ds(s * 3200, 3200)],
                    counts_out.at[c, pl.ds(s * 3200, 3200)])


@functools.lru_cache(maxsize=None)
def _sc_deg_gate():
    return pl.kernel(
        _sc_deg_gate_body,
        out_type=(jax.ShapeDtypeStruct((NC2, NCNT), _f32),
                  jax.ShapeDtypeStruct((E,), _f32)),
        mesh=_get_mesh(),
        compiler_params=pltpu.CompilerParams(use_tc_tiling_on_sc=False),
        scratch_types=(
            [pltpu.VMEM((_K0,), jnp.int32)] * 4
            + [pltpu.VMEM((_K0,), _f32)] * 6
            + [pltpu.VMEM_SHARED((NCNT,), _f32)]
            + [pltpu.SemaphoreType.DMA] * 8
        ),
    )


@functools.lru_cache(maxsize=None)
def _make_scatter(kind, K):
    nch = E // K

    def body(src, dst, table, *rest):
        if kind == "plain":
            (out, sidx0, sidx1, didx0, didx1, rows0, rows1, acc,
             sS0, sS1, sD0, sD1, sG0, sG1) = rest
            sE = ex = None
        elif kind == "erow":
            (erow, out, sidx0, sidx1, didx0, didx1, rows0, rows1,
             ex0, ex1, acc, sS0, sS1, sD0, sD1, sG0, sG1, sE0, sE1) = rest
            ex = (ex0, ex1)
            sE = (sE0, sE1)
        else:
            (gateh, out, sidx0, sidx1, didx0, didx1, rows0, rows1,
             ex0, ex1, acc, sS0, sS1, sD0, sD1, sG0, sG1, sE0, sE1) = rest
            ex = (ex0, ex1)
            sE = (sE0, sE1)
        c = lax.axis_index("c")
        s = lax.axis_index("s")
        w = s * NC2 + c
        sidx = (sidx0, sidx1)
        didx = (didx0, didx1)
        rows = (rows0, rows1)
        sS = (sS0, sS1)
        sD = (sD0, sD1)
        sG = (sG0, sG1)
        _zero_vmem_2d(rows0, K)
        _zero_acc(acc, rows0, K, s)
        plsc.subcore_barrier()
        T = (nch - w + NW - 1) // NW

        def base(j):
            return (w + NW * j) * K

        def issue_idx(b, j):
            pltpu.async_copy(src.at[pl.ds(base(j), K)], sidx[b], sS[b])
            pltpu.async_copy(dst.at[pl.ds(base(j), K)], didx[b], sD[b])

        def wait_idx(b):
            pltpu.make_async_copy(src.at[pl.ds(0, K)], sidx[b], sS[b]).wait()
            pltpu.make_async_copy(dst.at[pl.ds(0, K)], didx[b], sD[b]).wait()

        def issue_gather(b, j):
            pltpu.async_copy(table.at[sidx[b]], rows[b], sG[b])
            if kind == "erow":
                pltpu.async_copy(erow.at[pl.ds(base(j), K)], ex[b], sE[b])
            elif kind == "gate":
                pltpu.async_copy(gateh.at[pl.ds(base(j), K)], ex[b], sE[b])

        def wait_gather(b):
            pltpu.make_async_copy(table.at[sidx[b]], rows[b], sG[b]).wait()
            if kind == "erow":
                pltpu.make_async_copy(erow.at[pl.ds(0, K)], ex[b], sE[b]).wait()
            elif kind == "gate":
                pltpu.make_async_copy(gateh.at[pl.ds(0, K)], ex[b], sE[b]).wait()

        def consume(b, j):
            if kind == "gate":
                def rowfn(i, c2):
                    gv = ex[b][pl.ds(i * 16, 16)]
                    for l in range(16):
                        k = i * 16 + l
                        g = gv[l]
                        rows[b][k, pl.ds(0, 16)] = rows[b][k, pl.ds(0, 16)] * g
                        rows[b][k, pl.ds(16, 16)] = rows[b][k, pl.ds(16, 16)] * g
                    return c2
                lax.fori_loop(0, K // 16, rowfn, 0)
            pltpu.sync_copy(rows[b], acc.at[didx[b]], add=True)
            if kind == "erow":
                pltpu.sync_copy(ex[b], acc.at[didx[b]], add=True)

        _pipelined(T, issue_idx, wait_idx, issue_gather, wait_gather, consume)
        plsc.subcore_barrier()
        pltpu.sync_copy(acc.at[pl.ds(s * NPC, NPC)],
                        out.at[c, pl.ds(s * NPC, NPC)])

    scratch = [pltpu.VMEM((K,), jnp.int32)] * 4 + [pltpu.VMEM((K, H), _f32)] * 2
    nsem = 6
    if kind == "erow":
        scratch += [pltpu.VMEM((K, H), _f32)] * 2
        nsem = 8
    elif kind == "gate":
        scratch += [pltpu.VMEM((K,), _f32)] * 2
        nsem = 8
    scratch += [pltpu.VMEM_SHARED((NACC, H), _f32)]
    scratch += [pltpu.SemaphoreType.DMA] * nsem
    return pl.kernel(
        body,
        out_type=jax.ShapeDtypeStruct((NC2, NACC, H), _f32),
        mesh=_get_mesh(),
        compiler_params=pltpu.CompilerParams(use_tc_tiling_on_sc=False),
        scratch_types=scratch,
    )


def _sc_scatter_plain(*args):
    return _make_scatter("plain", 400)(*args)


def _sc_scatter_gate(*args):
    return _make_scatter("gate", 400)(*args)


_K3 = 128
_NCH3 = E // _K3


def _sc_escatter_body(src, dst, ta, tb, ch, out,
                      sidx0, sidx1, didx0, didx1, ra0, ra1, rb0, rb1,
                      cc0, cc1, acc,
                      sS0, sS1, sD0, sD1, sG0, sG1, sE0, sE1, sC0, sC1):
    c = lax.axis_index("c")
    s = lax.axis_index("s")
    w = s * NC2 + c
    sidx = (sidx0, sidx1)
    didx = (didx0, didx1)
    ra = (ra0, ra1)
    rb = (rb0, rb1)
    cc = (cc0, cc1)
    sS = (sS0, sS1)
    sD = (sD0, sD1)
    sG = (sG0, sG1)
    sE = (sE0, sE1)
    sC = (sC0, sC1)
    _zero_vmem_2d(ra0, _K3)
    _zero_acc(acc, ra0, _K3, s)
    plsc.subcore_barrier()
    T = (_NCH3 - w + NW - 1) // NW

    def base(j):
        return (w + NW * j) * _K3

    def issue_idx(b, j):
        pltpu.async_copy(src.at[pl.ds(base(j), _K3)], sidx[b], sS[b])
        pltpu.async_copy(dst.at[pl.ds(base(j), _K3)], didx[b], sD[b])

    def wait_idx(b):
        pltpu.make_async_copy(src.at[pl.ds(0, _K3)], sidx[b], sS[b]).wait()
        pltpu.make_async_copy(dst.at[pl.ds(0, _K3)], didx[b], sD[b]).wait()

    def issue_gather(b, j):
        pltpu.async_copy(ta.at[sidx[b]], ra[b], sG[b])
        pltpu.async_copy(tb.at[didx[b]], rb[b], sE[b])
        pltpu.async_copy(ch.at[pl.ds(base(j), _K3)], cc[b], sC[b])

    def wait_gather(b):
        pltpu.make_async_copy(ta.at[sidx[b]], ra[b], sG[b]).wait()
        pltpu.make_async_copy(tb.at[didx[b]], rb[b], sE[b]).wait()
        pltpu.make_async_copy(ch.at[pl.ds(0, _K3)], cc[b], sC[b]).wait()

    def consume(b, j):
        def efn(k, c2):
            for hh in (0, 16):
                v = (ra[b][k, pl.ds(hh, 16)] + rb[b][k, pl.ds(hh, 16)]
                     + cc[b][k, pl.ds(hh, 16)])
                ra[b][k, pl.ds(hh, 16)] = jnp.maximum(v, 0.0)
            return c2
        lax.fori_loop(0, _K3, efn, 0)
        pltpu.sync_copy(ra[b], acc.at[didx[b]], add=True)

    _pipelined(T, issue_idx, wait_idx, issue_gather, wait_gather, consume)
    plsc.subcore_barrier()
    pltpu.sync_copy(acc.at[pl.ds(s * NPC, NPC)],
                    out.at[c, pl.ds(s * NPC, NPC)])


@functools.lru_cache(maxsize=None)
def _sc_escatter():
    return pl.kernel(
        _sc_escatter_body,
        out_type=jax.ShapeDtypeStruct((NC2, NACC, H), _f32),
        mesh=_get_mesh(),
        compiler_params=pltpu.CompilerParams(use_tc_tiling_on_sc=False),
        scratch_types=(
            [pltpu.VMEM((_K3,), jnp.int32)] * 4
            + [pltpu.VMEM((_K3, H), _f32)] * 6
            + [pltpu.VMEM_SHARED((NACC, H), _f32)]
            + [pltpu.SemaphoreType.DMA] * 10
        ),
    )


def _rows(d, bn=BN):
    return pl.BlockSpec((bn, d), lambda i: (i, 0))


def _full(a):
    nd = a.ndim
    return pl.BlockSpec(a.shape, lambda i, _nd=nd: (0,) * _nd)


def _agg_spec(bn=BN):
    return pl.BlockSpec((NC2, bn, H), lambda i: (0, i, 0))


def _relu(x):
    return jnp.maximum(x, 0.0)


def _sigm(x):
    return 1.0 / (1.0 + jnp.exp(-x))


def _tc(body, in_specs, out_shapes, out_specs, grid):
    return pl.pallas_call(
        body,
        grid=(grid,),
        in_specs=in_specs,
        out_specs=out_specs,
        out_shape=out_shapes,
    )


def _sds(*shape):
    return jax.ShapeDtypeStruct(shape, _f32)


def kernel(discrete_x, continous_x, edge_index, edge_attr, churn_date, t, params):
    p = params
    src = edge_index[0]
    dst = edge_index[1]
    cd = churn_date[:, 0]
    cdp = cd * p['g1_wg'] + 0.5 * p['g1_bg']
    cdm = cd * p['g1_wg'] - 0.5 * p['g1_bg']

    counts2, gate = _sc_deg_gate()(src, dst, cdp, cdm)
    counts2 = counts2[:, :N, None]

    elW_A = p['el_W'][:H]
    elW_B = p['el_W'][H:2 * H]
    elW_C = p['el_W'][2 * H:]

    def tc1(disc, cont, cnt2, Wd, bd, Wc, bc, g0W, g0b, g1W,
            xd_o, xc_o, h1_o, hn1_o, norm_o, invdeg_o):
        cnt = cnt2[0] + cnt2[1]
        norm = lax.rsqrt(cnt + 1.0)
        invdeg = 1.0 / jnp.maximum(cnt, 1.0)
        norm_o[...] = norm
        invdeg_o[...] = invdeg
        xd = jnp.dot(disc[...], Wd[...], preferred_element_type=_f32) + bd[...]
        xd_o[...] = xd
        cont_v = cont[...]
        xcs = [_relu(jnp.dot(cont_v[:, 16 * i:16 * (i + 1)], Wc[...],
                             preferred_element_type=_f32) + bc[...])
               for i in range(3)]
        xc = jnp.concatenate(xcs, axis=-1)
        xc_o[...] = xc
        xg = _relu(jnp.dot(jnp.concatenate([xd, xc], axis=-1), g0W[...],
                           preferred_element_type=_f32) + g0b[...])
        h1 = jnp.dot(xg, g1W[...], preferred_element_type=_f32)
        h1_o[...] = h1
        hn1_o[...] = h1 * norm

    xd, xc, h1, hn1, normv, invdeg = _tc(
        tc1,
        [_rows(16), _rows(48), pl.BlockSpec((NC2, BN, 1), lambda i: (0, i, 0)),
         _full(p['W_d']), _full(p['b_d']), _full(p['W_c']), _full(p['b_c']),
         _full(p['g0_W']), _full(p['g0_b']), _full(p['gcn1_W'])],
        (_sds(N, 10), _sds(N, 24), _sds(N, H), _sds(N, H), _sds(N, 1), _sds(N, 1)),
        (_rows(10), _rows(24), _rows(H), _rows(H), _rows(1), _rows(1)),
        25,
    )(discrete_x, continous_x, counts2, p['W_d'], p['b_d'], p['W_c'], p['b_c'],
      p['g0_W'], p['g0_b'], p['gcn1_W'])

    agg1 = _sc_scatter_plain(src, dst, hn1)

    def tc2(agg, h1r, nr, g2W, b1, xg0_o, h2_o, hn2_o):
        nv = nr[...]
        xg0 = _relu((agg[0] + agg[1]) * nv + h1r[...] * nv * nv + b1[...])
        xg0_o[...] = xg0
        h2 = jnp.dot(xg0, g2W[...], preferred_element_type=_f32)
        h2_o[...] = h2
        hn2_o[...] = h2 * nv

    xg0, h2, hn2 = _tc(
        tc2,
        [_agg_spec(), _rows(H), _rows(1), _full(p['gcn2_W']), _full(p['gcn1_b'])],
        (_sds(N, H), _sds(N, H), _sds(N, H)),
        (_rows(H), _rows(H), _rows(H)),
        25,
    )(agg1, h1, normv, p['gcn2_W'], p['gcn1_b'])

    agg2 = _sc_scatter_plain(src, dst, hn2)

    def tc3(agg, h2r, nr, xdr, xcr, xg0r, b2,
            res1W, res1b, res2W, res2b, cr1w, cr1b, cr2w, cr2b, fuW, fub,
            si0W, si0b, eWA, eWB, el1Wx, el1Wr, el1b, c0W, c0b, g1Wm, g1Wr, g1b,
            hci_o, xsi_o, A_o, B_o, hx1_o, xr1b_o, xns_o, hm1_o, xrn1b_o):
        nv = nr[...]
        xg1 = _relu((agg[0] + agg[1]) * nv + h2r[...] * nv * nv + b2[...])
        x = jnp.concatenate([xdr[...], xcr[...], xg0r[...] + xg1], axis=-1)
        h1r = _relu(jnp.dot(x, res1W[...], preferred_element_type=_f32) + res1b[...]) + x
        x_deep = _relu(jnp.dot(h1r, res2W[...], preferred_element_type=_f32) + res2b[...]) + h1r
        xl = x
        s1 = jnp.dot(xl, cr1w[...], preferred_element_type=_f32)
        xl = x * s1 + cr1b[...] + xl
        s2 = jnp.dot(xl, cr2w[...], preferred_element_type=_f32)
        xl = x * s2 + cr2b[...] + xl
        hci_o[...] = _relu(jnp.dot(x_deep + xl, fuW[...], preferred_element_type=_f32) + fub[...])
        xsi = _relu(jnp.dot(x, si0W[...], preferred_element_type=_f32) + si0b[...])
        xsi_o[...] = xsi
        A_o[...] = jnp.dot(xsi, eWA[...], preferred_element_type=_f32)
        B_o[...] = jnp.dot(xsi, eWB[...], preferred_element_type=_f32)
        hx1_o[...] = jnp.dot(xsi, el1Wx[...], preferred_element_type=_f32)
        xr1b_o[...] = jnp.dot(xsi, el1Wr[...], preferred_element_type=_f32) + el1b[...]
        xns = _relu(jnp.dot(x, c0W[...], preferred_element_type=_f32) + c0b[...])
        xns_o[...] = xns
        hm1_o[...] = jnp.dot(xns, g1Wm[...], preferred_element_type=_f32)
        xrn1b_o[...] = jnp.dot(xns, g1Wr[...], preferred_element_type=_f32) + g1b[...]

    cr1w = p['cr1_w'][:, None]
    cr2w = p['cr2_w'][:, None]
    h_ci, x_si, A, B, hx1, xr1b, x_ns, hm1, xrn1b = _tc(
        tc3,
        [_agg_spec(), _rows(H), _rows(1), _rows(10), _rows(24), _rows(H),
         _full(p['gcn2_b']),
         _full(p['res1_W']), _full(p['res1_b']), _full(p['res2_W']), _full(p['res2_b']),
         _full(cr1w), _full(p['cr1_b']), _full(cr2w), _full(p['cr2_b']),
         _full(p['fu_W']), _full(p['fu_b']),
         _full(p['si0_W']), _full(p['si0_b']), _full(elW_A), _full(elW_B),
         _full(p['el1_Wx']), _full(p['el1_Wr']), _full(p['el1_b']),
         _full(p['c0_W']), _full(p['c0_b']),
         _full(p['g1_Wm']), _full(p['g1_Wr']), _full(p['g1_b'])],
        tuple(_sds(N, H) for _ in range(9)),
        tuple(_rows(H) for _ in range(9)),
        25,
    )(agg2, h2, normv, xd, xc, xg0, p['gcn2_b'],
      p['res1_W'], p['res1_b'], p['res2_W'], p['res2_b'],
      cr1w, p['cr1_b'], cr2w, p['cr2_b'], p['fu_W'], p['fu_b'],
      p['si0_W'], p['si0_b'], elW_A, elW_B,
      p['el1_Wx'], p['el1_Wr'], p['el1_b'], p['c0_W'], p['c0_b'],
      p['g1_Wm'], p['g1_Wr'], p['g1_b'])

    def tce1(ea, W, b, C_o):
        C_o[...] = jnp.dot(ea[...], W[...], preferred_element_type=_f32) + b[...]

    C = _tc(tce1, [_rows(16, BE), _full(elW_C), _full(p['el_b'])],
            _sds(E, H), _rows(H, BE), 100)(edge_attr, elW_C, p['el_b'])

    S_e = _sc_escatter()(src, dst, A, B, C)

    sagg1 = _sc_scatter_plain(src, dst, hx1)

    def tc4(agg, se, idg, xr1br, We1, el2Wx, el2Wr, el2b, xsi0_o, hx2_o, xr2b_o):
        eterm = jnp.dot(se[0] + se[1], We1[...], preferred_element_type=_f32)
        xsi0 = _relu((agg[0] + agg[1] + eterm) * idg[...] + xr1br[...])
        xsi0_o[...] = xsi0
        hx2_o[...] = jnp.dot(xsi0, el2Wx[...], preferred_element_type=_f32)
        xr2b_o[...] = jnp.dot(xsi0, el2Wr[...], preferred_element_type=_f32) + el2b[...]

    x_si0, hx2, xr2b = _tc(
        tc4,
        [_agg_spec(), _agg_spec(), _rows(1), _rows(H), _full(p['el1_We']),
         _full(p['el2_Wx']), _full(p['el2_Wr']), _full(p['el2_b'])],
        (_sds(N, H), _sds(N, H), _sds(N, H)), (_rows(H), _rows(H), _rows(H)), 25,
    )(sagg1, S_e, invdeg, xr1b, p['el1_We'],
      p['el2_Wx'], p['el2_Wr'], p['el2_b'])

    sagg2 = _sc_scatter_plain(src, dst, hx2)

    def tc5(agg, se, idg, xr2br, xsi0r, We2, TW, Tb, hsi_o, predT_o):
        eterm = jnp.dot(se[0] + se[1], We2[...], preferred_element_type=_f32)
        xsi1 = _relu((agg[0] + agg[1] + eterm) * idg[...] + xr2br[...])
        hsi = xsi0r[...] + xsi1
        hsi_o[...] = hsi
        predT_o[...] = _sigm(jnp.dot(hsi, TW[...], preferred_element_type=_f32) + Tb[...])

    h_si, pred_T = _tc(
        tc5,
        [_agg_spec(), _agg_spec(), _rows(1), _rows(H), _rows(H),
         _full(p['el2_We']), _full(p['T_W']), _full(p['T_b'])],
        (_sds(N, H), _sds(N, 1)), (_rows(H), _rows(1)), 25,
    )(sagg2, S_e, invdeg, xr2b, x_si0, p['el2_We'], p['T_W'], p['T_b'])

    gagg1 = _sc_scatter_gate(src, dst, hm1, gate)

    def tc6(agg, idg, xrn1br, g1Wm, g1Wr, g1b, xns0_o, hm2_o, xrn2b_o):
        xns0 = _relu((agg[0] + agg[1]) * idg[...] + xrn1br[...])
        xns0_o[...] = xns0
        hm2_o[...] = jnp.dot(xns0, g1Wm[...], preferred_element_type=_f32)
        xrn2b_o[...] = jnp.dot(xns0, g1Wr[...], preferred_element_type=_f32) + g1b[...]

    x_ns0, hm2, xrn2b = _tc(
        tc6,
        [_agg_spec(), _rows(1), _rows(H),
         _full(p['g1_Wm']), _full(p['g1_Wr']), _full(p['g1_b'])],
        (_sds(N, H), _sds(N, H), _sds(N, H)), (_rows(H), _rows(H), _rows(H)), 25,
    )(gagg1, invdeg, xrn1b, p['g1_Wm'], p['g1_Wr'], p['g1_b'])

    gagg2 = _sc_scatter_gate(src, dst, hm2, gate)

    def tc7(agg, idg, xrn2br, xns0r, hcir, hsir, tr,
            a0W, a0b, a1W, a1b, y0hW, y0hb, y0oW, y0ob, y1hW, y1hb, y1oW, y1ob,
            py_o, pycf_o, py0_o, py1_o):
        xns1 = _relu((agg[0] + agg[1]) * idg[...] + xrn2br[...])
        hns = xns0r[...] + xns1
        hci = hcir[...]
        hsi = hsir[...]
        h = jnp.concatenate([hci, hsi, hns], axis=-1)
        a0 = jax.nn.softmax(jnp.dot(h, a0W[...], preferred_element_type=_f32) + a0b[...], axis=-1)
        py0 = a0[:, :H] * hci + a0[:, H:2 * H] * hsi + a0[:, 2 * H:] * hns
        a1 = jax.nn.softmax(jnp.dot(h, a1W[...], preferred_element_type=_f32) + a1b[...], axis=-1)
        py1 = a1[:, :H] * hci + a1[:, H:2 * H] * hsi + a1[:, 2 * H:] * hns
        py0 = _sigm(jnp.dot(_relu(jnp.dot(py0, y0hW[...], preferred_element_type=_f32) + y0hb[...]),
                            y0oW[...], preferred_element_type=_f32) + y0ob[...])
        py1 = _sigm(jnp.dot(_relu(jnp.dot(py1, y1hW[...], preferred_element_type=_f32) + y1hb[...]),
                            y1oW[...], preferred_element_type=_f32) + y1ob[...])
        tv = tr[...]
        py_o[...] = (1.0 - tv) * py0 + tv * py1
        pycf_o[...] = tv * py0 + (1.0 - tv) * py1
        py0_o[...] = py0
        py1_o[...] = py1

    pred_y, pred_y_cf, pred_y0, pred_y1 = _tc(
        tc7,
        [_agg_spec(), _rows(1), _rows(H), _rows(H), _rows(H), _rows(H), _rows(1),
         _full(p['a0_W']), _full(p['a0_b']), _full(p['a1_W']), _full(p['a1_b']),
         _full(p['y0h_W']), _full(p['y0h_b']), _full(p['y0o_W']), _full(p['y0o_b']),
         _full(p['y1h_W']), _full(p['y1h_b']), _full(p['y1o_W']), _full(p['y1o_b'])],
        (_sds(N, 1), _sds(N, 1), _sds(N, 1), _sds(N, 1)),
        (_rows(1), _rows(1), _rows(1), _rows(1)),
        25,
    )(gagg2, invdeg, xrn2b, x_ns0, h_ci, h_si, t,
      p['a0_W'], p['a0_b'], p['a1_W'], p['a1_b'],
      p['y0h_W'], p['y0h_b'], p['y0o_W'], p['y0o_b'],
      p['y1h_W'], p['y1h_b'], p['y1o_W'], p['y1o_b'])

    return (pred_y, pred_y_cf, pred_y0, pred_y1, pred_T, h_ci, h_si)


# speedup vs baseline: 1.2482x; 1.0131x over previous
import functools

import jax
import jax.numpy as jnp
from jax import lax
from jax.experimental import pallas as pl
from jax.experimental.pallas import tpu as pltpu
from jax.experimental.pallas import tpu_sc as plsc

N = 50000
E = 800000
H = 32
NC2 = 2
NS = 16
NW = NC2 * NS
NACC = 50048
NPC = NACC // NS
ZR = 136
NCNT = 51200
BN = 2000
BE = 8000

_f32 = jnp.float32


@functools.lru_cache(maxsize=None)
def _get_mesh():
    return plsc.VectorSubcoreMesh(core_axis_name="c", subcore_axis_name="s",
                                  num_cores=NC2, num_subcores=NS)


def _zero_vmem_2d(ref, rows):
    def zrow(j, c):
        ref[j, pl.ds(0, 16)] = jnp.zeros((16,), _f32)
        ref[j, pl.ds(16, 16)] = jnp.zeros((16,), _f32)
        return c
    lax.fori_loop(0, rows, zrow, 0)


def _fill_vmem_1d(ref, n, val):
    def z16(i, c):
        ref[pl.ds(i * 16, 16)] = jnp.full((16,), val, _f32)
        return c
    lax.fori_loop(0, n // 16, z16, 0)


def _zero_acc(acc, zrow, K, s):
    q, r = NPC // K, NPC % K

    def zcp(j, c):
        pltpu.sync_copy(zrow, acc.at[pl.ds(s * NPC + j * K, K)])
        return c
    lax.fori_loop(0, q, zcp, 0)
    if r:
        pltpu.sync_copy(zrow.at[pl.ds(0, r)], acc.at[pl.ds(s * NPC + q * K, r)])


def _pipelined(T, issue_idx, wait_idx, issue_gather, wait_gather, consume):
    issue_idx(0, 0)
    wait_idx(0)
    issue_gather(0, 0)
    issue_idx(1, 1)

    def pair(jj, cc):
        j0 = 2 * jj
        j1 = j0 + 1
        wait_gather(0)

        @pl.when(j1 < T)
        def _():
            wait_idx(1)
            issue_gather(1, j1)
        consume(0, j0)

        @pl.when(j0 + 2 < T)
        def _():
            issue_idx(0, j0 + 2)

        @pl.when(j1 < T)
        def _():
            wait_gather(1)

            @pl.when(j1 + 1 < T)
            def _():
                wait_idx(0)
                issue_gather(0, j1 + 1)
            consume(1, j1)

            @pl.when(j1 + 2 < T)
            def _():
                issue_idx(1, j1 + 2)
        return cc
    lax.fori_loop(0, (T + 1) // 2, pair, 0)


_K0 = 1600
_NCH0 = E // _K0


def _sc_deg_gate_body(src, dst, cdp, cdm, counts_out, gate_out,
                      sidx0, sidx1, didx0, didx1, av0, av1, bv0, bv1,
                      onesv, zb1, acc1,
                      sS0, sS1, sD0, sD1, sG0, sG1, sE0, sE1):
    c = lax.axis_index("c")
    s = lax.axis_index("s")
    w = s * NC2 + c
    sidx = (sidx0, sidx1)
    didx = (didx0, didx1)
    av = (av0, av1)
    bv = (bv0, bv1)
    sS = (sS0, sS1)
    sD = (sD0, sD1)
    sG = (sG0, sG1)
    sE = (sE0, sE1)
    _fill_vmem_1d(zb1, _K0, 0.0)
    _fill_vmem_1d(onesv, _K0, 1.0)

    def zcp(j, cc):
        pltpu.sync_copy(zb1, acc1.at[pl.ds(s * 3200 + j * _K0, _K0)])
        return cc
    lax.fori_loop(0, 3200 // _K0, zcp, 0)
    plsc.subcore_barrier()
    T = (_NCH0 - w + NW - 1) // NW

    def base(j):
        return (w + NW * j) * _K0

    def issue_idx(b, j):
        pltpu.async_copy(src.at[pl.ds(base(j), _K0)], sidx[b], sS[b])
        pltpu.async_copy(dst.at[pl.ds(base(j), _K0)], didx[b], sD[b])

    def wait_idx(b):
        pltpu.make_async_copy(src.at[pl.ds(0, _K0)], sidx[b], sS[b]).wait()
        pltpu.make_async_copy(dst.at[pl.ds(0, _K0)], didx[b], sD[b]).wait()

    def issue_gather(b, j):
        pltpu.async_copy(cdp.at[sidx[b]], av[b], sG[b])
        pltpu.async_copy(cdm.at[didx[b]], bv[b], sE[b])

    def wait_gather(b):
        pltpu.make_async_copy(cdp.at[sidx[b]], av[b], sG[b]).wait()
        pltpu.make_async_copy(cdm.at[didx[b]], bv[b], sE[b]).wait()

    def consume(b, j):
        pltpu.sync_copy(onesv, acc1.at[didx[b]], add=True)

        def gfn(i, c2):
            z = av[b][pl.ds(i * 16, 16)] - bv[b][pl.ds(i * 16, 16)]
            av[b][pl.ds(i * 16, 16)] = 1.0 / (1.0 + jnp.exp(-z))
            return c2
        lax.fori_loop(0, _K0 // 16, gfn, 0)
        pltpu.sync_copy(av[b], gate_out.at[pl.ds(base(j), _K0)])

    _pipelined(T, issue_idx, wait_idx, issue_gather, wait_gather, consume)
    plsc.subcore_barrier()
    pltpu.sync_copy(acc1.at[pl.ds(s * 3200, 3200)],
                    counts_out.at[c, pl.ds(s * 3200, 3200)])


@functools.lru_cache(maxsize=None)
def _sc_deg_gate():
    return pl.kernel(
        _sc_deg_gate_body,
        out_type=(jax.ShapeDtypeStruct((NC2, NCNT), _f32),
                  jax.ShapeDtypeStruct((E,), _f32)),
        mesh=_get_mesh(),
        compiler_params=pltpu.CompilerParams(use_tc_tiling_on_sc=False),
        scratch_types=(
            [pltpu.VMEM((_K0,), jnp.int32)] * 4
            + [pltpu.VMEM((_K0,), _f32)] * 6
            + [pltpu.VMEM_SHARED((NCNT,), _f32)]
            + [pltpu.SemaphoreType.DMA] * 8
        ),
    )


@functools.lru_cache(maxsize=None)
def _make_scatter(kind, K):
    nch = E // K

    def body(src, dst, table, *rest):
        if kind == "plain":
            (out, sidx0, sidx1, didx0, didx1, rows0, rows1, acc,
             sS0, sS1, sD0, sD1, sG0, sG1) = rest
            sE = ex = None
        elif kind == "erow":
            (erow, out, sidx0, sidx1, didx0, didx1, rows0, rows1,
             ex0, ex1, acc, sS0, sS1, sD0, sD1, sG0, sG1, sE0, sE1) = rest
            ex = (ex0, ex1)
            sE = (sE0, sE1)
        else:
            (gateh, out, sidx0, sidx1, didx0, didx1, dS0, dS1, rows0, rows1,
             ex0, ex1, acc, sS0, sS1, sD0, sD1, sG0, sG1, sE0, sE1,
             sW0, sW1) = rest
            ex = (ex0, ex1)
            sE = (sE0, sE1)
            dS = (dS0, dS1)
            sW = (sW0, sW1)
        c = lax.axis_index("c")
        s = lax.axis_index("s")
        w = s * NC2 + c
        sidx = (sidx0, sidx1)
        didx = (didx0, didx1)
        rows = (rows0, rows1)
        sS = (sS0, sS1)
        sD = (sD0, sD1)
        sG = (sG0, sG1)
        _zero_vmem_2d(rows0, K)
        _zero_acc(acc, rows0, K, s)
        plsc.subcore_barrier()
        T = (nch - w + NW - 1) // NW

        def base(j):
            return (w + NW * j) * K

        def issue_idx(b, j):
            pltpu.async_copy(src.at[pl.ds(base(j), K)], sidx[b], sS[b])
            pltpu.async_copy(dst.at[pl.ds(base(j), K)], didx[b], sD[b])

        def wait_idx(b):
            pltpu.make_async_copy(src.at[pl.ds(0, K)], sidx[b], sS[b]).wait()
            pltpu.make_async_copy(dst.at[pl.ds(0, K)], didx[b], sD[b]).wait()

        def issue_gather(b, j):
            if kind == "gate":
                @pl.when(jnp.int32(j) >= 2)
                def _():
                    pltpu.make_async_copy(rows[b], acc.at[dS[b]], sW[b]).wait()
            pltpu.async_copy(table.at[sidx[b]], rows[b], sG[b])
            if kind == "erow":
                pltpu.async_copy(erow.at[pl.ds(base(j), K)], ex[b], sE[b])
            elif kind == "gate":
                pltpu.async_copy(gateh.at[pl.ds(base(j), K)], ex[b], sE[b])

        def wait_gather(b):
            pltpu.make_async_copy(table.at[sidx[b]], rows[b], sG[b]).wait()
            if kind == "erow":
                pltpu.make_async_copy(erow.at[pl.ds(0, K)], ex[b], sE[b]).wait()
            elif kind == "gate":
                pltpu.make_async_copy(gateh.at[pl.ds(0, K)], ex[b], sE[b]).wait()

        def consume(b, j):
            if kind == "gate":
                def rowfn(i, c2):
                    gv = ex[b][pl.ds(i * 16, 16)]
                    for l in range(16):
                        k = i * 16 + l
                        g = gv[l]
                        rows[b][k, pl.ds(0, 16)] = rows[b][k, pl.ds(0, 16)] * g
                        rows[b][k, pl.ds(16, 16)] = rows[b][k, pl.ds(16, 16)] * g
                    return c2
                lax.fori_loop(0, K // 16, rowfn, 0)
                def cidx(i, c2):
                    dS[b][pl.ds(i * 16, 16)] = didx[b][pl.ds(i * 16, 16)]
                    return c2
                lax.fori_loop(0, K // 16, cidx, 0)
                pltpu.async_copy(rows[b], acc.at[dS[b]], sW[b], add=True)
            else:
                pltpu.sync_copy(rows[b], acc.at[didx[b]], add=True)
            if kind == "erow":
                pltpu.sync_copy(ex[b], acc.at[didx[b]], add=True)

        _pipelined(T, issue_idx, wait_idx, issue_gather, wait_gather, consume)
        if kind == "gate":
            pltpu.make_async_copy(rows[0], acc.at[dS[0]], sW[0]).wait()
            pltpu.make_async_copy(rows[1], acc.at[dS[1]], sW[1]).wait()
        plsc.subcore_barrier()
        pltpu.sync_copy(acc.at[pl.ds(s * NPC, NPC)],
                        out.at[c, pl.ds(s * NPC, NPC)])

    if kind == "gate":
        scratch = ([pltpu.VMEM((K,), jnp.int32)] * 6
                   + [pltpu.VMEM((K, H), _f32)] * 2
                   + [pltpu.VMEM((K,), _f32)] * 2)
        nsem = 10
    else:
        scratch = ([pltpu.VMEM((K,), jnp.int32)] * 4
                   + [pltpu.VMEM((K, H), _f32)] * 2)
        nsem = 6
        if kind == "erow":
            scratch += [pltpu.VMEM((K, H), _f32)] * 2
            nsem = 8
    scratch += [pltpu.VMEM_SHARED((NACC, H), _f32)]
    scratch += [pltpu.SemaphoreType.DMA] * nsem
    return pl.kernel(
        body,
        out_type=jax.ShapeDtypeStruct((NC2, NACC, H), _f32),
        mesh=_get_mesh(),
        compiler_params=pltpu.CompilerParams(use_tc_tiling_on_sc=False),
        scratch_types=scratch,
    )


def _sc_scatter_plain(*args):
    return _make_scatter("plain", 400)(*args)


def _sc_scatter_gate(*args):
    return _make_scatter("gate", 400)(*args)


_K3 = 128
_NCH3 = E // _K3


def _sc_escatter_body(src, dst, ta, tb, ch, out,
                      sidx0, sidx1, didx0, didx1, ra0, ra1, rb0, rb1,
                      cc0, cc1, acc,
                      sS0, sS1, sD0, sD1, sG0, sG1, sE0, sE1, sC0, sC1):
    c = lax.axis_index("c")
    s = lax.axis_index("s")
    w = s * NC2 + c
    sidx = (sidx0, sidx1)
    didx = (didx0, didx1)
    ra = (ra0, ra1)
    rb = (rb0, rb1)
    cc = (cc0, cc1)
    sS = (sS0, sS1)
    sD = (sD0, sD1)
    sG = (sG0, sG1)
    sE = (sE0, sE1)
    sC = (sC0, sC1)
    _zero_vmem_2d(ra0, _K3)
    _zero_acc(acc, ra0, _K3, s)
    plsc.subcore_barrier()
    T = (_NCH3 - w + NW - 1) // NW

    def base(j):
        return (w + NW * j) * _K3

    def issue_idx(b, j):
        pltpu.async_copy(src.at[pl.ds(base(j), _K3)], sidx[b], sS[b])
        pltpu.async_copy(dst.at[pl.ds(base(j), _K3)], didx[b], sD[b])

    def wait_idx(b):
        pltpu.make_async_copy(src.at[pl.ds(0, _K3)], sidx[b], sS[b]).wait()
        pltpu.make_async_copy(dst.at[pl.ds(0, _K3)], didx[b], sD[b]).wait()

    def issue_gather(b, j):
        pltpu.async_copy(ta.at[sidx[b]], ra[b], sG[b])
        pltpu.async_copy(tb.at[didx[b]], rb[b], sE[b])
        pltpu.async_copy(ch.at[pl.ds(base(j), _K3)], cc[b], sC[b])

    def wait_gather(b):
        pltpu.make_async_copy(ta.at[sidx[b]], ra[b], sG[b]).wait()
        pltpu.make_async_copy(tb.at[didx[b]], rb[b], sE[b]).wait()
        pltpu.make_async_copy(ch.at[pl.ds(0, _K3)], cc[b], sC[b]).wait()

    def consume(b, j):
        def efn(k, c2):
            for hh in (0, 16):
                v = (ra[b][k, pl.ds(hh, 16)] + rb[b][k, pl.ds(hh, 16)]
                     + cc[b][k, pl.ds(hh, 16)])
                ra[b][k, pl.ds(hh, 16)] = jnp.maximum(v, 0.0)
            return c2
        lax.fori_loop(0, _K3, efn, 0)
        pltpu.sync_copy(ra[b], acc.at[didx[b]], add=True)

    _pipelined(T, issue_idx, wait_idx, issue_gather, wait_gather, consume)
    plsc.subcore_barrier()
    pltpu.sync_copy(acc.at[pl.ds(s * NPC, NPC)],
                    out.at[c, pl.ds(s * NPC, NPC)])


@functools.lru_cache(maxsize=None)
def _sc_escatter():
    return pl.kernel(
        _sc_escatter_body,
        out_type=jax.ShapeDtypeStruct((NC2, NACC, H), _f32),
        mesh=_get_mesh(),
        compiler_params=pltpu.CompilerParams(use_tc_tiling_on_sc=False),
        scratch_types=(
            [pltpu.VMEM((_K3,), jnp.int32)] * 4
            + [pltpu.VMEM((_K3, H), _f32)] * 6
            + [pltpu.VMEM_SHARED((NACC, H), _f32)]
            + [pltpu.SemaphoreType.DMA] * 10
        ),
    )


def _rows(d, bn=BN):
    return pl.BlockSpec((bn, d), lambda i: (i, 0))


def _full(a):
    nd = a.ndim
    return pl.BlockSpec(a.shape, lambda i, _nd=nd: (0,) * _nd)


def _agg_spec(bn=BN):
    return pl.BlockSpec((NC2, bn, H), lambda i: (0, i, 0))


def _relu(x):
    return jnp.maximum(x, 0.0)


def _sigm(x):
    return 1.0 / (1.0 + jnp.exp(-x))


def _tc(body, in_specs, out_shapes, out_specs, grid):
    return pl.pallas_call(
        body,
        grid=(grid,),
        in_specs=in_specs,
        out_specs=out_specs,
        out_shape=out_shapes,
    )


def _sds(*shape):
    return jax.ShapeDtypeStruct(shape, _f32)


def kernel(discrete_x, continous_x, edge_index, edge_attr, churn_date, t, params):
    p = params
    src = edge_index[0]
    dst = edge_index[1]
    cd = churn_date[:, 0]
    cdp = cd * p['g1_wg'] + 0.5 * p['g1_bg']
    cdm = cd * p['g1_wg'] - 0.5 * p['g1_bg']

    counts2, gate = _sc_deg_gate()(src, dst, cdp, cdm)
    counts2 = counts2[:, :N, None]

    elW_A = p['el_W'][:H]
    elW_B = p['el_W'][H:2 * H]
    elW_C = p['el_W'][2 * H:]

    def tc1(disc, cont, cnt2, Wd, bd, Wc, bc, g0W, g0b, g1W,
            xd_o, xc_o, h1_o, hn1_o, norm_o, invdeg_o):
        cnt = cnt2[0] + cnt2[1]
        norm = lax.rsqrt(cnt + 1.0)
        invdeg = 1.0 / jnp.maximum(cnt, 1.0)
        norm_o[...] = norm
        invdeg_o[...] = invdeg
        xd = jnp.dot(disc[...], Wd[...], preferred_element_type=_f32) + bd[...]
        xd_o[...] = xd
        cont_v = cont[...]
        xcs = [_relu(jnp.dot(cont_v[:, 16 * i:16 * (i + 1)], Wc[...],
                             preferred_element_type=_f32) + bc[...])
               for i in range(3)]
        xc = jnp.concatenate(xcs, axis=-1)
        xc_o[...] = xc
        xg = _relu(jnp.dot(jnp.concatenate([xd, xc], axis=-1), g0W[...],
                           preferred_element_type=_f32) + g0b[...])
        h1 = jnp.dot(xg, g1W[...], preferred_element_type=_f32)
        h1_o[...] = h1
        hn1_o[...] = h1 * norm

    xd, xc, h1, hn1, normv, invdeg = _tc(
        tc1,
        [_rows(16), _rows(48), pl.BlockSpec((NC2, BN, 1), lambda i: (0, i, 0)),
         _full(p['W_d']), _full(p['b_d']), _full(p['W_c']), _full(p['b_c']),
         _full(p['g0_W']), _full(p['g0_b']), _full(p['gcn1_W'])],
        (_sds(N, 10), _sds(N, 24), _sds(N, H), _sds(N, H), _sds(N, 1), _sds(N, 1)),
        (_rows(10), _rows(24), _rows(H), _rows(H), _rows(1), _rows(1)),
        25,
    )(discrete_x, continous_x, counts2, p['W_d'], p['b_d'], p['W_c'], p['b_c'],
      p['g0_W'], p['g0_b'], p['gcn1_W'])

    agg1 = _sc_scatter_plain(src, dst, hn1)

    def tc2(agg, h1r, nr, g2W, b1, xg0_o, h2_o, hn2_o):
        nv = nr[...]
        xg0 = _relu((agg[0] + agg[1]) * nv + h1r[...] * nv * nv + b1[...])
        xg0_o[...] = xg0
        h2 = jnp.dot(xg0, g2W[...], preferred_element_type=_f32)
        h2_o[...] = h2
        hn2_o[...] = h2 * nv

    xg0, h2, hn2 = _tc(
        tc2,
        [_agg_spec(), _rows(H), _rows(1), _full(p['gcn2_W']), _full(p['gcn1_b'])],
        (_sds(N, H), _sds(N, H), _sds(N, H)),
        (_rows(H), _rows(H), _rows(H)),
        25,
    )(agg1, h1, normv, p['gcn2_W'], p['gcn1_b'])

    agg2 = _sc_scatter_plain(src, dst, hn2)

    def tc3(agg, h2r, nr, xdr, xcr, xg0r, b2,
            res1W, res1b, res2W, res2b, cr1w, cr1b, cr2w, cr2b, fuW, fub,
            si0W, si0b, eWA, eWB, el1Wx, el1Wr, el1b, c0W, c0b, g1Wm, g1Wr, g1b,
            hci_o, xsi_o, A_o, B_o, hx1_o, xr1b_o, xns_o, hm1_o, xrn1b_o):
        nv = nr[...]
        xg1 = _relu((agg[0] + agg[1]) * nv + h2r[...] * nv * nv + b2[...])
        x = jnp.concatenate([xdr[...], xcr[...], xg0r[...] + xg1], axis=-1)
        h1r = _relu(jnp.dot(x, res1W[...], preferred_element_type=_f32) + res1b[...]) + x
        x_deep = _relu(jnp.dot(h1r, res2W[...], preferred_element_type=_f32) + res2b[...]) + h1r
        xl = x
        s1 = jnp.dot(xl, cr1w[...], preferred_element_type=_f32)
        xl = x * s1 + cr1b[...] + xl
        s2 = jnp.dot(xl, cr2w[...], preferred_element_type=_f32)
        xl = x * s2 + cr2b[...] + xl
        hci_o[...] = _relu(jnp.dot(x_deep + xl, fuW[...], preferred_element_type=_f32) + fub[...])
        xsi = _relu(jnp.dot(x, si0W[...], preferred_element_type=_f32) + si0b[...])
        xsi_o[...] = xsi
        A_o[...] = jnp.dot(xsi, eWA[...], preferred_element_type=_f32)
        B_o[...] = jnp.dot(xsi, eWB[...], preferred_element_type=_f32)
        hx1_o[...] = jnp.dot(xsi, el1Wx[...], preferred_element_type=_f32)
        xr1b_o[...] = jnp.dot(xsi, el1Wr[...], preferred_element_type=_f32) + el1b[...]
        xns = _relu(jnp.dot(x, c0W[...], preferred_element_type=_f32) + c0b[...])
        xns_o[...] = xns
        hm1_o[...] = jnp.dot(xns, g1Wm[...], preferred_element_type=_f32)
        xrn1b_o[...] = jnp.dot(xns, g1Wr[...], preferred_element_type=_f32) + g1b[...]

    cr1w = p['cr1_w'][:, None]
    cr2w = p['cr2_w'][:, None]
    h_ci, x_si, A, B, hx1, xr1b, x_ns, hm1, xrn1b = _tc(
        tc3,
        [_agg_spec(), _rows(H), _rows(1), _rows(10), _rows(24), _rows(H),
         _full(p['gcn2_b']),
         _full(p['res1_W']), _full(p['res1_b']), _full(p['res2_W']), _full(p['res2_b']),
         _full(cr1w), _full(p['cr1_b']), _full(cr2w), _full(p['cr2_b']),
         _full(p['fu_W']), _full(p['fu_b']),
         _full(p['si0_W']), _full(p['si0_b']), _full(elW_A), _full(elW_B),
         _full(p['el1_Wx']), _full(p['el1_Wr']), _full(p['el1_b']),
         _full(p['c0_W']), _full(p['c0_b']),
         _full(p['g1_Wm']), _full(p['g1_Wr']), _full(p['g1_b'])],
        tuple(_sds(N, H) for _ in range(9)),
        tuple(_rows(H) for _ in range(9)),
        25,
    )(agg2, h2, normv, xd, xc, xg0, p['gcn2_b'],
      p['res1_W'], p['res1_b'], p['res2_W'], p['res2_b'],
      cr1w, p['cr1_b'], cr2w, p['cr2_b'], p['fu_W'], p['fu_b'],
      p['si0_W'], p['si0_b'], elW_A, elW_B,
      p['el1_Wx'], p['el1_Wr'], p['el1_b'], p['c0_W'], p['c0_b'],
      p['g1_Wm'], p['g1_Wr'], p['g1_b'])

    def tce1(ea, W, b, C_o):
        C_o[...] = jnp.dot(ea[...], W[...], preferred_element_type=_f32) + b[...]

    C = _tc(tce1, [_rows(16, BE), _full(elW_C), _full(p['el_b'])],
            _sds(E, H), _rows(H, BE), 100)(edge_attr, elW_C, p['el_b'])

    S_e = _sc_escatter()(src, dst, A, B, C)

    sagg1 = _sc_scatter_plain(src, dst, hx1)

    def tc4(agg, se, idg, xr1br, We1, el2Wx, el2Wr, el2b, xsi0_o, hx2_o, xr2b_o):
        eterm = jnp.dot(se[0] + se[1], We1[...], preferred_element_type=_f32)
        xsi0 = _relu((agg[0] + agg[1] + eterm) * idg[...] + xr1br[...])
        xsi0_o[...] = xsi0
        hx2_o[...] = jnp.dot(xsi0, el2Wx[...], preferred_element_type=_f32)
        xr2b_o[...] = jnp.dot(xsi0, el2Wr[...], preferred_element_type=_f32) + el2b[...]

    x_si0, hx2, xr2b = _tc(
        tc4,
        [_agg_spec(), _agg_spec(), _rows(1), _rows(H), _full(p['el1_We']),
         _full(p['el2_Wx']), _full(p['el2_Wr']), _full(p['el2_b'])],
        (_sds(N, H), _sds(N, H), _sds(N, H)), (_rows(H), _rows(H), _rows(H)), 25,
    )(sagg1, S_e, invdeg, xr1b, p['el1_We'],
      p['el2_Wx'], p['el2_Wr'], p['el2_b'])

    sagg2 = _sc_scatter_plain(src, dst, hx2)

    def tc5(agg, se, idg, xr2br, xsi0r, We2, TW, Tb, hsi_o, predT_o):
        eterm = jnp.dot(se[0] + se[1], We2[...], preferred_element_type=_f32)
        xsi1 = _relu((agg[0] + agg[1] + eterm) * idg[...] + xr2br[...])
        hsi = xsi0r[...] + xsi1
        hsi_o[...] = hsi
        predT_o[...] = _sigm(jnp.dot(hsi, TW[...], preferred_element_type=_f32) + Tb[...])

    h_si, pred_T = _tc(
        tc5,
        [_agg_spec(), _agg_spec(), _rows(1), _rows(H), _rows(H),
         _full(p['el2_We']), _full(p['T_W']), _full(p['T_b'])],
        (_sds(N, H), _sds(N, 1)), (_rows(H), _rows(1)), 25,
    )(sagg2, S_e, invdeg, xr2b, x_si0, p['el2_We'], p['T_W'], p['T_b'])

    gagg1 = _sc_scatter_gate(src, dst, hm1, gate)

    def tc6(agg, idg, xrn1br, g1Wm, g1Wr, g1b, xns0_o, hm2_o, xrn2b_o):
        xns0 = _relu((agg[0] + agg[1]) * idg[...] + xrn1br[...])
        xns0_o[...] = xns0
        hm2_o[...] = jnp.dot(xns0, g1Wm[...], preferred_element_type=_f32)
        xrn2b_o[...] = jnp.dot(xns0, g1Wr[...], preferred_element_type=_f32) + g1b[...]

    x_ns0, hm2, xrn2b = _tc(
        tc6,
        [_agg_spec(), _rows(1), _rows(H),
         _full(p['g1_Wm']), _full(p['g1_Wr']), _full(p['g1_b'])],
        (_sds(N, H), _sds(N, H), _sds(N, H)), (_rows(H), _rows(H), _rows(H)), 25,
    )(gagg1, invdeg, xrn1b, p['g1_Wm'], p['g1_Wr'], p['g1_b'])

    gagg2 = _sc_scatter_gate(src, dst, hm2, gate)

    def tc7(agg, idg, xrn2br, xns0r, hcir, hsir, tr,
            a0W, a0b, a1W, a1b, y0hW, y0hb, y0oW, y0ob, y1hW, y1hb, y1oW, y1ob,
            py_o, pycf_o, py0_o, py1_o):
        xns1 = _relu((agg[0] + agg[1]) * idg[...] + xrn2br[...])
        hns = xns0r[...] + xns1
        hci = hcir[...]
        hsi = hsir[...]
        h = jnp.concatenate([hci, hsi, hns], axis=-1)
        a0 = jax.nn.softmax(jnp.dot(h, a0W[...], preferred_element_type=_f32) + a0b[...], axis=-1)
        py0 = a0[:, :H] * hci + a0[:, H:2 * H] * hsi + a0[:, 2 * H:] * hns
        a1 = jax.nn.softmax(jnp.dot(h, a1W[...], preferred_element_type=_f32) + a1b[...], axis=-1)
        py1 = a1[:, :H] * hci + a1[:, H:2 * H] * hsi + a1[:, 2 * H:] * hns
        py0 = _sigm(jnp.dot(_relu(jnp.dot(py0, y0hW[...], preferred_element_type=_f32) + y0hb[...]),
                            y0oW[...], preferred_element_type=_f32) + y0ob[...])
        py1 = _sigm(jnp.dot(_relu(jnp.dot(py1, y1hW[...], preferred_element_type=_f32) + y1hb[...]),
                            y1oW[...], preferred_element_type=_f32) + y1ob[...])
        tv = tr[...]
        py_o[...] = (1.0 - tv) * py0 + tv * py1
        pycf_o[...] = tv * py0 + (1.0 - tv) * py1
        py0_o[...] = py0
        py1_o[...] = py1

    pred_y, pred_y_cf, pred_y0, pred_y1 = _tc(
        tc7,
        [_agg_spec(), _rows(1), _rows(H), _rows(H), _rows(H), _rows(H), _rows(1),
         _full(p['a0_W']), _full(p['a0_b']), _full(p['a1_W']), _full(p['a1_b']),
         _full(p['y0h_W']), _full(p['y0h_b']), _full(p['y0o_W']), _full(p['y0o_b']),
         _full(p['y1h_W']), _full(p['y1h_b']), _full(p['y1o_W']), _full(p['y1o_b'])],
        (_sds(N, 1), _sds(N, 1), _sds(N, 1), _sds(N, 1)),
        (_rows(1), _rows(1), _rows(1), _rows(1)),
        25,
    )(gagg2, invdeg, xrn2b, x_ns0, h_ci, h_si, t,
      p['a0_W'], p['a0_b'], p['a1_W'], p['a1_b'],
      p['y0h_W'], p['y0h_b'], p['y0o_W'], p['y0o_b'],
      p['y1h_W'], p['y1h_b'], p['y1o_W'], p['y1o_b'])

    return (pred_y, pred_y_cf, pred_y0, pred_y1, pred_T, h_ci, h_si)


# speedup vs baseline: 1.2935x; 1.0363x over previous
import functools

import jax
import jax.numpy as jnp
from jax import lax
from jax.experimental import pallas as pl
from jax.experimental.pallas import tpu as pltpu
from jax.experimental.pallas import tpu_sc as plsc

N = 50000
E = 800000
H = 32
NC2 = 2
NS = 16
NW = NC2 * NS
NACC = 50048
NPC = NACC // NS
ZR = 136
NCNT = 51200
BN = 2000
BE = 8000

_f32 = jnp.float32


@functools.lru_cache(maxsize=None)
def _get_mesh():
    return plsc.VectorSubcoreMesh(core_axis_name="c", subcore_axis_name="s",
                                  num_cores=NC2, num_subcores=NS)


def _zero_vmem_2d(ref, rows):
    def zrow(j, c):
        ref[j, pl.ds(0, 16)] = jnp.zeros((16,), _f32)
        ref[j, pl.ds(16, 16)] = jnp.zeros((16,), _f32)
        return c
    lax.fori_loop(0, rows, zrow, 0)


def _fill_vmem_1d(ref, n, val):
    def z16(i, c):
        ref[pl.ds(i * 16, 16)] = jnp.full((16,), val, _f32)
        return c
    lax.fori_loop(0, n // 16, z16, 0)


def _zero_acc(acc, zrow, K, s):
    q, r = NPC // K, NPC % K

    def zcp(j, c):
        pltpu.sync_copy(zrow, acc.at[pl.ds(s * NPC + j * K, K)])
        return c
    lax.fori_loop(0, q, zcp, 0)
    if r:
        pltpu.sync_copy(zrow.at[pl.ds(0, r)], acc.at[pl.ds(s * NPC + q * K, r)])


def _pipelined(T, issue_idx, wait_idx, issue_gather, wait_gather, consume):
    issue_idx(0, 0)
    wait_idx(0)
    issue_gather(0, 0)
    issue_idx(1, 1)

    def pair(jj, cc):
        j0 = 2 * jj
        j1 = j0 + 1
        wait_gather(0)

        @pl.when(j1 < T)
        def _():
            wait_idx(1)
            issue_gather(1, j1)
        consume(0, j0)

        @pl.when(j0 + 2 < T)
        def _():
            issue_idx(0, j0 + 2)

        @pl.when(j1 < T)
        def _():
            wait_gather(1)

            @pl.when(j1 + 1 < T)
            def _():
                wait_idx(0)
                issue_gather(0, j1 + 1)
            consume(1, j1)

            @pl.when(j1 + 2 < T)
            def _():
                issue_idx(1, j1 + 2)
        return cc
    lax.fori_loop(0, (T + 1) // 2, pair, 0)


_K0 = 1600
_NCH0 = E // _K0


def _sc_deg_gate_body(src, dst, cdp, cdm, counts_out, gate_out,
                      sidx0, sidx1, didx0, didx1, av0, av1, bv0, bv1,
                      onesv, zb1, acc1,
                      sS0, sS1, sD0, sD1, sG0, sG1, sE0, sE1):
    c = lax.axis_index("c")
    s = lax.axis_index("s")
    w = s * NC2 + c
    sidx = (sidx0, sidx1)
    didx = (didx0, didx1)
    av = (av0, av1)
    bv = (bv0, bv1)
    sS = (sS0, sS1)
    sD = (sD0, sD1)
    sG = (sG0, sG1)
    sE = (sE0, sE1)
    _fill_vmem_1d(zb1, _K0, 0.0)
    _fill_vmem_1d(onesv, _K0, 1.0)

    def zcp(j, cc):
        pltpu.sync_copy(zb1, acc1.at[pl.ds(s * 3200 + j * _K0, _K0)])
        return cc
    lax.fori_loop(0, 3200 // _K0, zcp, 0)
    plsc.subcore_barrier()
    T = (_NCH0 - w + NW - 1) // NW

    def base(j):
        return (w + NW * j) * _K0

    def issue_idx(b, j):
        pltpu.async_copy(src.at[pl.ds(base(j), _K0)], sidx[b], sS[b])
        pltpu.async_copy(dst.at[pl.ds(base(j), _K0)], didx[b], sD[b])

    def wait_idx(b):
        pltpu.make_async_copy(src.at[pl.ds(0, _K0)], sidx[b], sS[b]).wait()
        pltpu.make_async_copy(dst.at[pl.ds(0, _K0)], didx[b], sD[b]).wait()

    def issue_gather(b, j):
        pltpu.async_copy(cdp.at[sidx[b]], av[b], sG[b])
        pltpu.async_copy(cdm.at[didx[b]], bv[b], sE[b])

    def wait_gather(b):
        pltpu.make_async_copy(cdp.at[sidx[b]], av[b], sG[b]).wait()
        pltpu.make_async_copy(cdm.at[didx[b]], bv[b], sE[b]).wait()

    def consume(b, j):
        pltpu.sync_copy(onesv, acc1.at[didx[b]], add=True)

        def gfn(i, c2):
            z = av[b][pl.ds(i * 16, 16)] - bv[b][pl.ds(i * 16, 16)]
            av[b][pl.ds(i * 16, 16)] = 1.0 / (1.0 + jnp.exp(-z))
            return c2
        lax.fori_loop(0, _K0 // 16, gfn, 0)
        pltpu.sync_copy(av[b], gate_out.at[pl.ds(base(j), _K0)])

    _pipelined(T, issue_idx, wait_idx, issue_gather, wait_gather, consume)
    plsc.subcore_barrier()
    pltpu.sync_copy(acc1.at[pl.ds(s * 3200, 3200)],
                    counts_out.at[c, pl.ds(s * 3200, 3200)])


@functools.lru_cache(maxsize=None)
def _sc_deg_gate():
    return pl.kernel(
        _sc_deg_gate_body,
        out_type=(jax.ShapeDtypeStruct((NC2, NCNT), _f32),
                  jax.ShapeDtypeStruct((E,), _f32)),
        mesh=_get_mesh(),
        compiler_params=pltpu.CompilerParams(use_tc_tiling_on_sc=False),
        scratch_types=(
            [pltpu.VMEM((_K0,), jnp.int32)] * 4
            + [pltpu.VMEM((_K0,), _f32)] * 6
            + [pltpu.VMEM_SHARED((NCNT,), _f32)]
            + [pltpu.SemaphoreType.DMA] * 8
        ),
    )


@functools.lru_cache(maxsize=None)
def _make_scatter(kind, K):
    nch = E // K

    def body(src, dst, table, *rest):
        if kind == "plain":
            (out, sidx0, sidx1, didx0, didx1, rows0, rows1, acc,
             sS0, sS1, sD0, sD1, sG0, sG1) = rest
            sE = ex = None
        elif kind == "erow":
            (erow, out, sidx0, sidx1, didx0, didx1, rows0, rows1,
             ex0, ex1, acc, sS0, sS1, sD0, sD1, sG0, sG1, sE0, sE1) = rest
            ex = (ex0, ex1)
            sE = (sE0, sE1)
        else:
            (gateh, out, sidx0, sidx1, didx0, didx1, dS0, dS1, rows0, rows1,
             ex0, ex1, acc, sS0, sS1, sD0, sD1, sG0, sG1, sE0, sE1,
             sW0, sW1) = rest
            ex = (ex0, ex1)
            sE = (sE0, sE1)
            dS = (dS0, dS1)
            sW = (sW0, sW1)
        c = lax.axis_index("c")
        s = lax.axis_index("s")
        w = s * NC2 + c
        sidx = (sidx0, sidx1)
        didx = (didx0, didx1)
        rows = (rows0, rows1)
        sS = (sS0, sS1)
        sD = (sD0, sD1)
        sG = (sG0, sG1)
        _zero_vmem_2d(rows0, K)
        _zero_acc(acc, rows0, K, s)
        plsc.subcore_barrier()
        T = (nch - w + NW - 1) // NW

        def base(j):
            return (w + NW * j) * K

        def issue_idx(b, j):
            pltpu.async_copy(src.at[pl.ds(base(j), K)], sidx[b], sS[b])
            pltpu.async_copy(dst.at[pl.ds(base(j), K)], didx[b], sD[b])

        def wait_idx(b):
            pltpu.make_async_copy(src.at[pl.ds(0, K)], sidx[b], sS[b]).wait()
            pltpu.make_async_copy(dst.at[pl.ds(0, K)], didx[b], sD[b]).wait()

        def issue_gather(b, j):
            if kind == "gate":
                @pl.when(jnp.int32(j) >= 2)
                def _():
                    pltpu.make_async_copy(rows[b], acc.at[dS[b]], sW[b]).wait()
            pltpu.async_copy(table.at[sidx[b]], rows[b], sG[b])
            if kind == "erow":
                pltpu.async_copy(erow.at[pl.ds(base(j), K)], ex[b], sE[b])
            elif kind == "gate":
                pltpu.async_copy(gateh.at[pl.ds(base(j), K)], ex[b], sE[b])

        def wait_gather(b):
            pltpu.make_async_copy(table.at[sidx[b]], rows[b], sG[b]).wait()
            if kind == "erow":
                pltpu.make_async_copy(erow.at[pl.ds(0, K)], ex[b], sE[b]).wait()
            elif kind == "gate":
                pltpu.make_async_copy(gateh.at[pl.ds(0, K)], ex[b], sE[b]).wait()

        def consume(b, j):
            if kind == "gate":
                def rowfn(i, c2):
                    gv = ex[b][pl.ds(i * 16, 16)]
                    for l in range(16):
                        k = i * 16 + l
                        g = gv[l]
                        rows[b][k, pl.ds(0, 16)] = rows[b][k, pl.ds(0, 16)] * g
                        rows[b][k, pl.ds(16, 16)] = rows[b][k, pl.ds(16, 16)] * g
                    return c2
                lax.fori_loop(0, K // 16, rowfn, 0)
                def cidx(i, c2):
                    dS[b][pl.ds(i * 16, 16)] = didx[b][pl.ds(i * 16, 16)]
                    return c2
                lax.fori_loop(0, K // 16, cidx, 0)
                pltpu.async_copy(rows[b], acc.at[dS[b]], sW[b], add=True)
            else:
                pltpu.sync_copy(rows[b], acc.at[didx[b]], add=True)
            if kind == "erow":
                pltpu.sync_copy(ex[b], acc.at[didx[b]], add=True)

        _pipelined(T, issue_idx, wait_idx, issue_gather, wait_gather, consume)
        if kind == "gate":
            pltpu.make_async_copy(rows[0], acc.at[dS[0]], sW[0]).wait()
            pltpu.make_async_copy(rows[1], acc.at[dS[1]], sW[1]).wait()
        plsc.subcore_barrier()
        pltpu.sync_copy(acc.at[pl.ds(s * NPC, NPC)],
                        out.at[c, pl.ds(s * NPC, NPC)])

    if kind == "gate":
        scratch = ([pltpu.VMEM((K,), jnp.int32)] * 6
                   + [pltpu.VMEM((K, H), _f32)] * 2
                   + [pltpu.VMEM((K,), _f32)] * 2)
        nsem = 10
    else:
        scratch = ([pltpu.VMEM((K,), jnp.int32)] * 4
                   + [pltpu.VMEM((K, H), _f32)] * 2)
        nsem = 6
        if kind == "erow":
            scratch += [pltpu.VMEM((K, H), _f32)] * 2
            nsem = 8
    scratch += [pltpu.VMEM_SHARED((NACC, H), _f32)]
    scratch += [pltpu.SemaphoreType.DMA] * nsem
    return pl.kernel(
        body,
        out_type=jax.ShapeDtypeStruct((NC2, NACC, H), _f32),
        mesh=_get_mesh(),
        compiler_params=pltpu.CompilerParams(use_tc_tiling_on_sc=False),
        scratch_types=scratch,
    )


def _sc_scatter_plain(*args):
    return _make_scatter("plain", 400)(*args)


def _sc_scatter_gate(*args):
    return _make_scatter("gate", 400)(*args)


_K3 = 128
_NCH3 = E // _K3


def _sc_escatter_body(src, dst, ta, tb, ch, out,
                      sidx0, sidx1, didx0, didx1, dS0, dS1, ra0, ra1, rb0, rb1,
                      cc0, cc1, acc,
                      sS0, sS1, sD0, sD1, sG0, sG1, sE0, sE1, sC0, sC1,
                      sW0, sW1):
    c = lax.axis_index("c")
    s = lax.axis_index("s")
    w = s * NC2 + c
    sidx = (sidx0, sidx1)
    didx = (didx0, didx1)
    dS = (dS0, dS1)
    ra = (ra0, ra1)
    rb = (rb0, rb1)
    cc = (cc0, cc1)
    sS = (sS0, sS1)
    sD = (sD0, sD1)
    sG = (sG0, sG1)
    sE = (sE0, sE1)
    sC = (sC0, sC1)
    sW = (sW0, sW1)
    _zero_vmem_2d(ra0, _K3)
    _zero_acc(acc, ra0, _K3, s)
    plsc.subcore_barrier()
    T = (_NCH3 - w + NW - 1) // NW

    def base(j):
        return (w + NW * j) * _K3

    def issue_idx(b, j):
        pltpu.async_copy(src.at[pl.ds(base(j), _K3)], sidx[b], sS[b])
        pltpu.async_copy(dst.at[pl.ds(base(j), _K3)], didx[b], sD[b])

    def wait_idx(b):
        pltpu.make_async_copy(src.at[pl.ds(0, _K3)], sidx[b], sS[b]).wait()
        pltpu.make_async_copy(dst.at[pl.ds(0, _K3)], didx[b], sD[b]).wait()

    def issue_gather(b, j):
        @pl.when(jnp.int32(j) >= 2)
        def _():
            pltpu.make_async_copy(ra[b], acc.at[dS[b]], sW[b]).wait()
        pltpu.async_copy(ta.at[sidx[b]], ra[b], sG[b])
        pltpu.async_copy(tb.at[didx[b]], rb[b], sE[b])
        pltpu.async_copy(ch.at[pl.ds(base(j), _K3)], cc[b], sC[b])

    def wait_gather(b):
        pltpu.make_async_copy(ta.at[sidx[b]], ra[b], sG[b]).wait()
        pltpu.make_async_copy(tb.at[didx[b]], rb[b], sE[b]).wait()
        pltpu.make_async_copy(ch.at[pl.ds(0, _K3)], cc[b], sC[b]).wait()

    def consume(b, j):
        def efn(i, c2):
            for k2 in range(2):
                k = i * 2 + k2
                for hh in (0, 16):
                    v = (ra[b][k, pl.ds(hh, 16)] + rb[b][k, pl.ds(hh, 16)]
                         + cc[b][k, pl.ds(hh, 16)])
                    ra[b][k, pl.ds(hh, 16)] = jnp.maximum(v, 0.0)
            return c2
        lax.fori_loop(0, _K3 // 2, efn, 0)

        def cidx(i, c2):
            dS[b][pl.ds(i * 16, 16)] = didx[b][pl.ds(i * 16, 16)]
            return c2
        lax.fori_loop(0, _K3 // 16, cidx, 0)
        pltpu.async_copy(ra[b], acc.at[dS[b]], sW[b], add=True)

    _pipelined(T, issue_idx, wait_idx, issue_gather, wait_gather, consume)
    pltpu.make_async_copy(ra[0], acc.at[dS[0]], sW[0]).wait()
    pltpu.make_async_copy(ra[1], acc.at[dS[1]], sW[1]).wait()
    plsc.subcore_barrier()
    pltpu.sync_copy(acc.at[pl.ds(s * NPC, NPC)],
                    out.at[c, pl.ds(s * NPC, NPC)])


@functools.lru_cache(maxsize=None)
def _sc_escatter():
    return pl.kernel(
        _sc_escatter_body,
        out_type=jax.ShapeDtypeStruct((NC2, NACC, H), _f32),
        mesh=_get_mesh(),
        compiler_params=pltpu.CompilerParams(use_tc_tiling_on_sc=False),
        scratch_types=(
            [pltpu.VMEM((_K3,), jnp.int32)] * 6
            + [pltpu.VMEM((_K3, H), _f32)] * 6
            + [pltpu.VMEM_SHARED((NACC, H), _f32)]
            + [pltpu.SemaphoreType.DMA] * 12
        ),
    )


def _rows(d, bn=BN):
    return pl.BlockSpec((bn, d), lambda i: (i, 0))


def _full(a):
    nd = a.ndim
    return pl.BlockSpec(a.shape, lambda i, _nd=nd: (0,) * _nd)


def _agg_spec(bn=BN):
    return pl.BlockSpec((NC2, bn, H), lambda i: (0, i, 0))


def _relu(x):
    return jnp.maximum(x, 0.0)


def _sigm(x):
    return 1.0 / (1.0 + jnp.exp(-x))


def _tc(body, in_specs, out_shapes, out_specs, grid):
    return pl.pallas_call(
        body,
        grid=(grid,),
        in_specs=in_specs,
        out_specs=out_specs,
        out_shape=out_shapes,
    )


def _sds(*shape):
    return jax.ShapeDtypeStruct(shape, _f32)


def kernel(discrete_x, continous_x, edge_index, edge_attr, churn_date, t, params):
    p = params
    src = edge_index[0]
    dst = edge_index[1]
    cd = churn_date[:, 0]
    cdp = cd * p['g1_wg'] + 0.5 * p['g1_bg']
    cdm = cd * p['g1_wg'] - 0.5 * p['g1_bg']

    counts2, gate = _sc_deg_gate()(src, dst, cdp, cdm)
    counts2 = counts2[:, :N, None]

    elW_A = p['el_W'][:H]
    elW_B = p['el_W'][H:2 * H]
    elW_C = p['el_W'][2 * H:]

    def tc1(disc, cont, cnt2, Wd, bd, Wc, bc, g0W, g0b, g1W,
            xd_o, xc_o, h1_o, hn1_o, norm_o, invdeg_o):
        cnt = cnt2[0] + cnt2[1]
        norm = lax.rsqrt(cnt + 1.0)
        invdeg = 1.0 / jnp.maximum(cnt, 1.0)
        norm_o[...] = norm
        invdeg_o[...] = invdeg
        xd = jnp.dot(disc[...], Wd[...], preferred_element_type=_f32) + bd[...]
        xd_o[...] = xd
        cont_v = cont[...]
        xcs = [_relu(jnp.dot(cont_v[:, 16 * i:16 * (i + 1)], Wc[...],
                             preferred_element_type=_f32) + bc[...])
               for i in range(3)]
        xc = jnp.concatenate(xcs, axis=-1)
        xc_o[...] = xc
        xg = _relu(jnp.dot(jnp.concatenate([xd, xc], axis=-1), g0W[...],
                           preferred_element_type=_f32) + g0b[...])
        h1 = jnp.dot(xg, g1W[...], preferred_element_type=_f32)
        h1_o[...] = h1
        hn1_o[...] = h1 * norm

    xd, xc, h1, hn1, normv, invdeg = _tc(
        tc1,
        [_rows(16), _rows(48), pl.BlockSpec((NC2, BN, 1), lambda i: (0, i, 0)),
         _full(p['W_d']), _full(p['b_d']), _full(p['W_c']), _full(p['b_c']),
         _full(p['g0_W']), _full(p['g0_b']), _full(p['gcn1_W'])],
        (_sds(N, 10), _sds(N, 24), _sds(N, H), _sds(N, H), _sds(N, 1), _sds(N, 1)),
        (_rows(10), _rows(24), _rows(H), _rows(H), _rows(1), _rows(1)),
        25,
    )(discrete_x, continous_x, counts2, p['W_d'], p['b_d'], p['W_c'], p['b_c'],
      p['g0_W'], p['g0_b'], p['gcn1_W'])

    agg1 = _sc_scatter_plain(src, dst, hn1)

    def tc2(agg, h1r, nr, g2W, b1, xg0_o, h2_o, hn2_o):
        nv = nr[...]
        xg0 = _relu((agg[0] + agg[1]) * nv + h1r[...] * nv * nv + b1[...])
        xg0_o[...] = xg0
        h2 = jnp.dot(xg0, g2W[...], preferred_element_type=_f32)
        h2_o[...] = h2
        hn2_o[...] = h2 * nv

    xg0, h2, hn2 = _tc(
        tc2,
        [_agg_spec(), _rows(H), _rows(1), _full(p['gcn2_W']), _full(p['gcn1_b'])],
        (_sds(N, H), _sds(N, H), _sds(N, H)),
        (_rows(H), _rows(H), _rows(H)),
        25,
    )(agg1, h1, normv, p['gcn2_W'], p['gcn1_b'])

    agg2 = _sc_scatter_plain(src, dst, hn2)

    def tc3(agg, h2r, nr, xdr, xcr, xg0r, b2,
            res1W, res1b, res2W, res2b, cr1w, cr1b, cr2w, cr2b, fuW, fub,
            si0W, si0b, eWA, eWB, el1Wx, el1Wr, el1b, c0W, c0b, g1Wm, g1Wr, g1b,
            hci_o, xsi_o, A_o, B_o, hx1_o, xr1b_o, xns_o, hm1_o, xrn1b_o):
        nv = nr[...]
        xg1 = _relu((agg[0] + agg[1]) * nv + h2r[...] * nv * nv + b2[...])
        x = jnp.concatenate([xdr[...], xcr[...], xg0r[...] + xg1], axis=-1)
        h1r = _relu(jnp.dot(x, res1W[...], preferred_element_type=_f32) + res1b[...]) + x
        x_deep = _relu(jnp.dot(h1r, res2W[...], preferred_element_type=_f32) + res2b[...]) + h1r
        xl = x
        s1 = jnp.dot(xl, cr1w[...], preferred_element_type=_f32)
        xl = x * s1 + cr1b[...] + xl
        s2 = jnp.dot(xl, cr2w[...], preferred_element_type=_f32)
        xl = x * s2 + cr2b[...] + xl
        hci_o[...] = _relu(jnp.dot(x_deep + xl, fuW[...], preferred_element_type=_f32) + fub[...])
        xsi = _relu(jnp.dot(x, si0W[...], preferred_element_type=_f32) + si0b[...])
        xsi_o[...] = xsi
        A_o[...] = jnp.dot(xsi, eWA[...], preferred_element_type=_f32)
        B_o[...] = jnp.dot(xsi, eWB[...], preferred_element_type=_f32)
        hx1_o[...] = jnp.dot(xsi, el1Wx[...], preferred_element_type=_f32)
        xr1b_o[...] = jnp.dot(xsi, el1Wr[...], preferred_element_type=_f32) + el1b[...]
        xns = _relu(jnp.dot(x, c0W[...], preferred_element_type=_f32) + c0b[...])
        xns_o[...] = xns
        hm1_o[...] = jnp.dot(xns, g1Wm[...], preferred_element_type=_f32)
        xrn1b_o[...] = jnp.dot(xns, g1Wr[...], preferred_element_type=_f32) + g1b[...]

    cr1w = p['cr1_w'][:, None]
    cr2w = p['cr2_w'][:, None]
    h_ci, x_si, A, B, hx1, xr1b, x_ns, hm1, xrn1b = _tc(
        tc3,
        [_agg_spec(), _rows(H), _rows(1), _rows(10), _rows(24), _rows(H),
         _full(p['gcn2_b']),
         _full(p['res1_W']), _full(p['res1_b']), _full(p['res2_W']), _full(p['res2_b']),
         _full(cr1w), _full(p['cr1_b']), _full(cr2w), _full(p['cr2_b']),
         _full(p['fu_W']), _full(p['fu_b']),
         _full(p['si0_W']), _full(p['si0_b']), _full(elW_A), _full(elW_B),
         _full(p['el1_Wx']), _full(p['el1_Wr']), _full(p['el1_b']),
         _full(p['c0_W']), _full(p['c0_b']),
         _full(p['g1_Wm']), _full(p['g1_Wr']), _full(p['g1_b'])],
        tuple(_sds(N, H) for _ in range(9)),
        tuple(_rows(H) for _ in range(9)),
        25,
    )(agg2, h2, normv, xd, xc, xg0, p['gcn2_b'],
      p['res1_W'], p['res1_b'], p['res2_W'], p['res2_b'],
      cr1w, p['cr1_b'], cr2w, p['cr2_b'], p['fu_W'], p['fu_b'],
      p['si0_W'], p['si0_b'], elW_A, elW_B,
      p['el1_Wx'], p['el1_Wr'], p['el1_b'], p['c0_W'], p['c0_b'],
      p['g1_Wm'], p['g1_Wr'], p['g1_b'])

    def tce1(ea, W, b, C_o):
        C_o[...] = jnp.dot(ea[...], W[...], preferred_element_type=_f32) + b[...]

    C = _tc(tce1, [_rows(16, BE), _full(elW_C), _full(p['el_b'])],
            _sds(E, H), _rows(H, BE), 100)(edge_attr, elW_C, p['el_b'])

    S_e = _sc_escatter()(src, dst, A, B, C)

    sagg1 = _sc_scatter_plain(src, dst, hx1)

    def tc4(agg, se, idg, xr1br, We1, el2Wx, el2Wr, el2b, xsi0_o, hx2_o, xr2b_o):
        eterm = jnp.dot(se[0] + se[1], We1[...], preferred_element_type=_f32)
        xsi0 = _relu((agg[0] + agg[1] + eterm) * idg[...] + xr1br[...])
        xsi0_o[...] = xsi0
        hx2_o[...] = jnp.dot(xsi0, el2Wx[...], preferred_element_type=_f32)
        xr2b_o[...] = jnp.dot(xsi0, el2Wr[...], preferred_element_type=_f32) + el2b[...]

    x_si0, hx2, xr2b = _tc(
        tc4,
        [_agg_spec(), _agg_spec(), _rows(1), _rows(H), _full(p['el1_We']),
         _full(p['el2_Wx']), _full(p['el2_Wr']), _full(p['el2_b'])],
        (_sds(N, H), _sds(N, H), _sds(N, H)), (_rows(H), _rows(H), _rows(H)), 25,
    )(sagg1, S_e, invdeg, xr1b, p['el1_We'],
      p['el2_Wx'], p['el2_Wr'], p['el2_b'])

    sagg2 = _sc_scatter_plain(src, dst, hx2)

    def tc5(agg, se, idg, xr2br, xsi0r, We2, TW, Tb, hsi_o, predT_o):
        eterm = jnp.dot(se[0] + se[1], We2[...], preferred_element_type=_f32)
        xsi1 = _relu((agg[0] + agg[1] + eterm) * idg[...] + xr2br[...])
        hsi = xsi0r[...] + xsi1
        hsi_o[...] = hsi
        predT_o[...] = _sigm(jnp.dot(hsi, TW[...], preferred_element_type=_f32) + Tb[...])

    h_si, pred_T = _tc(
        tc5,
        [_agg_spec(), _agg_spec(), _rows(1), _rows(H), _rows(H),
         _full(p['el2_We']), _full(p['T_W']), _full(p['T_b'])],
        (_sds(N, H), _sds(N, 1)), (_rows(H), _rows(1)), 25,
    )(sagg2, S_e, invdeg, xr2b, x_si0, p['el2_We'], p['T_W'], p['T_b'])

    gagg1 = _sc_scatter_gate(src, dst, hm1, gate)

    def tc6(agg, idg, xrn1br, g1Wm, g1Wr, g1b, xns0_o, hm2_o, xrn2b_o):
        xns0 = _relu((agg[0] + agg[1]) * idg[...] + xrn1br[...])
        xns0_o[...] = xns0
        hm2_o[...] = jnp.dot(xns0, g1Wm[...], preferred_element_type=_f32)
        xrn2b_o[...] = jnp.dot(xns0, g1Wr[...], preferred_element_type=_f32) + g1b[...]

    x_ns0, hm2, xrn2b = _tc(
        tc6,
        [_agg_spec(), _rows(1), _rows(H),
         _full(p['g1_Wm']), _full(p['g1_Wr']), _full(p['g1_b'])],
        (_sds(N, H), _sds(N, H), _sds(N, H)), (_rows(H), _rows(H), _rows(H)), 25,
    )(gagg1, invdeg, xrn1b, p['g1_Wm'], p['g1_Wr'], p['g1_b'])

    gagg2 = _sc_scatter_gate(src, dst, hm2, gate)

    def tc7(agg, idg, xrn2br, xns0r, hcir, hsir, tr,
            a0W, a0b, a1W, a1b, y0hW, y0hb, y0oW, y0ob, y1hW, y1hb, y1oW, y1ob,
            py_o, pycf_o, py0_o, py1_o):
        xns1 = _relu((agg[0] + agg[1]) * idg[...] + xrn2br[...])
        hns = xns0r[...] + xns1
        hci = hcir[...]
        hsi = hsir[...]
        h = jnp.concatenate([hci, hsi, hns], axis=-1)
        a0 = jax.nn.softmax(jnp.dot(h, a0W[...], preferred_element_type=_f32) + a0b[...], axis=-1)
        py0 = a0[:, :H] * hci + a0[:, H:2 * H] * hsi + a0[:, 2 * H:] * hns
        a1 = jax.nn.softmax(jnp.dot(h, a1W[...], preferred_element_type=_f32) + a1b[...], axis=-1)
        py1 = a1[:, :H] * hci + a1[:, H:2 * H] * hsi + a1[:, 2 * H:] * hns
        py0 = _sigm(jnp.dot(_relu(jnp.dot(py0, y0hW[...], preferred_element_type=_f32) + y0hb[...]),
                            y0oW[...], preferred_element_type=_f32) + y0ob[...])
        py1 = _sigm(jnp.dot(_relu(jnp.dot(py1, y1hW[...], preferred_element_type=_f32) + y1hb[...]),
                            y1oW[...], preferred_element_type=_f32) + y1ob[...])
        tv = tr[...]
        py_o[...] = (1.0 - tv) * py0 + tv * py1
        pycf_o[...] = tv * py0 + (1.0 - tv) * py1
        py0_o[...] = py0
        py1_o[...] = py1

    pred_y, pred_y_cf, pred_y0, pred_y1 = _tc(
        tc7,
        [_agg_spec(), _rows(1), _rows(H), _rows(H), _rows(H), _rows(H), _rows(1),
         _full(p['a0_W']), _full(p['a0_b']), _full(p['a1_W']), _full(p['a1_b']),
         _full(p['y0h_W']), _full(p['y0h_b']), _full(p['y0o_W']), _full(p['y0o_b']),
         _full(p['y1h_W']), _full(p['y1h_b']), _full(p['y1o_W']), _full(p['y1o_b'])],
        (_sds(N, 1), _sds(N, 1), _sds(N, 1), _sds(N, 1)),
        (_rows(1), _rows(1), _rows(1), _rows(1)),
        25,
    )(gagg2, invdeg, xrn2b, x_ns0, h_ci, h_si, t,
      p['a0_W'], p['a0_b'], p['a1_W'], p['a1_b'],
      p['y0h_W'], p['y0h_b'], p['y0o_W'], p['y0o_b'],
      p['y1h_W'], p['y1h_b'], p['y1o_W'], p['y1o_b'])

    return (pred_y, pred_y_cf, pred_y0, pred_y1, pred_T, h_ci, h_si)


# speedup vs baseline: 1.2941x; 1.0004x over previous
import functools

import jax
import jax.numpy as jnp
from jax import lax
from jax.experimental import pallas as pl
from jax.experimental.pallas import tpu as pltpu
from jax.experimental.pallas import tpu_sc as plsc

N = 50000
E = 800000
H = 32
NC2 = 2
NS = 16
NW = NC2 * NS
NACC = 50048
NPC = NACC // NS
ZR = 136
NCNT = 51200
BN = 2000
BE = 8000

_f32 = jnp.float32


@functools.lru_cache(maxsize=None)
def _get_mesh():
    return plsc.VectorSubcoreMesh(core_axis_name="c", subcore_axis_name="s",
                                  num_cores=NC2, num_subcores=NS)


def _zero_vmem_2d(ref, rows):
    def zrow(j, c):
        ref[j, pl.ds(0, 16)] = jnp.zeros((16,), _f32)
        ref[j, pl.ds(16, 16)] = jnp.zeros((16,), _f32)
        return c
    lax.fori_loop(0, rows, zrow, 0)


def _fill_vmem_1d(ref, n, val):
    def z16(i, c):
        ref[pl.ds(i * 16, 16)] = jnp.full((16,), val, _f32)
        return c
    lax.fori_loop(0, n // 16, z16, 0)


def _zero_acc(acc, zrow, K, s):
    q, r = NPC // K, NPC % K

    def zcp(j, c):
        pltpu.sync_copy(zrow, acc.at[pl.ds(s * NPC + j * K, K)])
        return c
    lax.fori_loop(0, q, zcp, 0)
    if r:
        pltpu.sync_copy(zrow.at[pl.ds(0, r)], acc.at[pl.ds(s * NPC + q * K, r)])


def _pipelined(T, issue_idx, wait_idx, issue_gather, wait_gather, consume):
    issue_idx(0, 0)
    wait_idx(0)
    issue_gather(0, 0)
    issue_idx(1, 1)

    def pair(jj, cc):
        j0 = 2 * jj
        j1 = j0 + 1
        wait_gather(0)

        @pl.when(j1 < T)
        def _():
            wait_idx(1)
            issue_gather(1, j1)
        consume(0, j0)

        @pl.when(j0 + 2 < T)
        def _():
            issue_idx(0, j0 + 2)

        @pl.when(j1 < T)
        def _():
            wait_gather(1)

            @pl.when(j1 + 1 < T)
            def _():
                wait_idx(0)
                issue_gather(0, j1 + 1)
            consume(1, j1)

            @pl.when(j1 + 2 < T)
            def _():
                issue_idx(1, j1 + 2)
        return cc
    lax.fori_loop(0, (T + 1) // 2, pair, 0)


_K0 = 1600
_NCH0 = E // _K0


def _sc_deg_gate_body(src, dst, cdp, cdm, counts_out, gate_out,
                      sidx0, sidx1, didx0, didx1, av0, av1, bv0, bv1,
                      onesv, zb1, acc1,
                      sS0, sS1, sD0, sD1, sG0, sG1, sE0, sE1):
    c = lax.axis_index("c")
    s = lax.axis_index("s")
    w = s * NC2 + c
    sidx = (sidx0, sidx1)
    didx = (didx0, didx1)
    av = (av0, av1)
    bv = (bv0, bv1)
    sS = (sS0, sS1)
    sD = (sD0, sD1)
    sG = (sG0, sG1)
    sE = (sE0, sE1)
    _fill_vmem_1d(zb1, _K0, 0.0)
    _fill_vmem_1d(onesv, _K0, 1.0)

    def zcp(j, cc):
        pltpu.sync_copy(zb1, acc1.at[pl.ds(s * 3200 + j * _K0, _K0)])
        return cc
    lax.fori_loop(0, 3200 // _K0, zcp, 0)
    plsc.subcore_barrier()
    T = (_NCH0 - w + NW - 1) // NW

    def base(j):
        return (w + NW * j) * _K0

    def issue_idx(b, j):
        pltpu.async_copy(src.at[pl.ds(base(j), _K0)], sidx[b], sS[b])
        pltpu.async_copy(dst.at[pl.ds(base(j), _K0)], didx[b], sD[b])

    def wait_idx(b):
        pltpu.make_async_copy(src.at[pl.ds(0, _K0)], sidx[b], sS[b]).wait()
        pltpu.make_async_copy(dst.at[pl.ds(0, _K0)], didx[b], sD[b]).wait()

    def issue_gather(b, j):
        pltpu.async_copy(cdp.at[sidx[b]], av[b], sG[b])
        pltpu.async_copy(cdm.at[didx[b]], bv[b], sE[b])

    def wait_gather(b):
        pltpu.make_async_copy(cdp.at[sidx[b]], av[b], sG[b]).wait()
        pltpu.make_async_copy(cdm.at[didx[b]], bv[b], sE[b]).wait()

    def consume(b, j):
        pltpu.sync_copy(onesv, acc1.at[didx[b]], add=True)

        def gfn(i, c2):
            z = av[b][pl.ds(i * 16, 16)] - bv[b][pl.ds(i * 16, 16)]
            av[b][pl.ds(i * 16, 16)] = 1.0 / (1.0 + jnp.exp(-z))
            return c2
        lax.fori_loop(0, _K0 // 16, gfn, 0)
        pltpu.sync_copy(av[b], gate_out.at[pl.ds(base(j), _K0)])

    _pipelined(T, issue_idx, wait_idx, issue_gather, wait_gather, consume)
    plsc.subcore_barrier()
    pltpu.sync_copy(acc1.at[pl.ds(s * 3200, 3200)],
                    counts_out.at[c, pl.ds(s * 3200, 3200)])


@functools.lru_cache(maxsize=None)
def _sc_deg_gate():
    return pl.kernel(
        _sc_deg_gate_body,
        out_type=(jax.ShapeDtypeStruct((NC2, NCNT), _f32),
                  jax.ShapeDtypeStruct((E,), _f32)),
        mesh=_get_mesh(),
        compiler_params=pltpu.CompilerParams(use_tc_tiling_on_sc=False),
        scratch_types=(
            [pltpu.VMEM((_K0,), jnp.int32)] * 4
            + [pltpu.VMEM((_K0,), _f32)] * 6
            + [pltpu.VMEM_SHARED((NCNT,), _f32)]
            + [pltpu.SemaphoreType.DMA] * 8
        ),
    )


@functools.lru_cache(maxsize=None)
def _make_scatter(kind, K):
    nch = E // K

    def body(src, dst, table, *rest):
        if kind == "plain":
            (out, sidx0, sidx1, didx0, didx1, dS0, dS1, rows0, rows1, acc,
             sS0, sS1, sD0, sD1, sG0, sG1, sW0, sW1) = rest
            sE = ex = None
        else:
            (gateh, out, sidx0, sidx1, didx0, didx1, dS0, dS1, rows0, rows1,
             ex0, ex1, acc, sS0, sS1, sD0, sD1, sG0, sG1, sE0, sE1,
             sW0, sW1) = rest
            ex = (ex0, ex1)
            sE = (sE0, sE1)
        dS = (dS0, dS1)
        sW = (sW0, sW1)
        c = lax.axis_index("c")
        s = lax.axis_index("s")
        w = s * NC2 + c
        sidx = (sidx0, sidx1)
        didx = (didx0, didx1)
        rows = (rows0, rows1)
        sS = (sS0, sS1)
        sD = (sD0, sD1)
        sG = (sG0, sG1)
        _zero_vmem_2d(rows0, K)
        _zero_acc(acc, rows0, K, s)
        plsc.subcore_barrier()
        T = (nch - w + NW - 1) // NW

        def base(j):
            return (w + NW * j) * K

        def issue_idx(b, j):
            pltpu.async_copy(src.at[pl.ds(base(j), K)], sidx[b], sS[b])
            pltpu.async_copy(dst.at[pl.ds(base(j), K)], didx[b], sD[b])

        def wait_idx(b):
            pltpu.make_async_copy(src.at[pl.ds(0, K)], sidx[b], sS[b]).wait()
            pltpu.make_async_copy(dst.at[pl.ds(0, K)], didx[b], sD[b]).wait()

        def issue_gather(b, j):
            @pl.when(jnp.int32(j) >= 2)
            def _():
                pltpu.make_async_copy(rows[b], acc.at[dS[b]], sW[b]).wait()
            pltpu.async_copy(table.at[sidx[b]], rows[b], sG[b])
            if kind == "gate":
                pltpu.async_copy(gateh.at[pl.ds(base(j), K)], ex[b], sE[b])

        def wait_gather(b):
            pltpu.make_async_copy(table.at[sidx[b]], rows[b], sG[b]).wait()
            if kind == "gate":
                pltpu.make_async_copy(gateh.at[pl.ds(0, K)], ex[b], sE[b]).wait()

        def consume(b, j):
            if kind == "gate":
                def rowfn(i, c2):
                    gv = ex[b][pl.ds(i * 16, 16)]
                    for l in range(16):
                        k = i * 16 + l
                        g = gv[l]
                        rows[b][k, pl.ds(0, 16)] = rows[b][k, pl.ds(0, 16)] * g
                        rows[b][k, pl.ds(16, 16)] = rows[b][k, pl.ds(16, 16)] * g
                    return c2
                lax.fori_loop(0, K // 16, rowfn, 0)
            def cidx(i, c2):
                dS[b][pl.ds(i * 16, 16)] = didx[b][pl.ds(i * 16, 16)]
                return c2
            lax.fori_loop(0, K // 16, cidx, 0)
            pltpu.async_copy(rows[b], acc.at[dS[b]], sW[b], add=True)

        _pipelined(T, issue_idx, wait_idx, issue_gather, wait_gather, consume)
        pltpu.make_async_copy(rows[0], acc.at[dS[0]], sW[0]).wait()
        pltpu.make_async_copy(rows[1], acc.at[dS[1]], sW[1]).wait()
        plsc.subcore_barrier()
        pltpu.sync_copy(acc.at[pl.ds(s * NPC, NPC)],
                        out.at[c, pl.ds(s * NPC, NPC)])

    scratch = ([pltpu.VMEM((K,), jnp.int32)] * 6
               + [pltpu.VMEM((K, H), _f32)] * 2)
    nsem = 8
    if kind == "gate":
        scratch += [pltpu.VMEM((K,), _f32)] * 2
        nsem = 10
    scratch += [pltpu.VMEM_SHARED((NACC, H), _f32)]
    scratch += [pltpu.SemaphoreType.DMA] * nsem
    return pl.kernel(
        body,
        out_type=jax.ShapeDtypeStruct((NC2, NACC, H), _f32),
        mesh=_get_mesh(),
        compiler_params=pltpu.CompilerParams(use_tc_tiling_on_sc=False),
        scratch_types=scratch,
    )


def _sc_scatter_plain(*args):
    return _make_scatter("plain", 400)(*args)


def _sc_scatter_gate(*args):
    return _make_scatter("gate", 400)(*args)


_K3 = 128
_NCH3 = E // _K3


def _sc_escatter_body(src, dst, ta, tb, ch, out,
                      sidx0, sidx1, didx0, didx1, dS0, dS1, ra0, ra1, rb0, rb1,
                      cc0, cc1, acc,
                      sS0, sS1, sD0, sD1, sG0, sG1, sE0, sE1, sC0, sC1,
                      sW0, sW1):
    c = lax.axis_index("c")
    s = lax.axis_index("s")
    w = s * NC2 + c
    sidx = (sidx0, sidx1)
    didx = (didx0, didx1)
    dS = (dS0, dS1)
    ra = (ra0, ra1)
    rb = (rb0, rb1)
    cc = (cc0, cc1)
    sS = (sS0, sS1)
    sD = (sD0, sD1)
    sG = (sG0, sG1)
    sE = (sE0, sE1)
    sC = (sC0, sC1)
    sW = (sW0, sW1)
    _zero_vmem_2d(ra0, _K3)
    _zero_acc(acc, ra0, _K3, s)
    plsc.subcore_barrier()
    T = (_NCH3 - w + NW - 1) // NW

    def base(j):
        return (w + NW * j) * _K3

    def issue_idx(b, j):
        pltpu.async_copy(src.at[pl.ds(base(j), _K3)], sidx[b], sS[b])
        pltpu.async_copy(dst.at[pl.ds(base(j), _K3)], didx[b], sD[b])

    def wait_idx(b):
        pltpu.make_async_copy(src.at[pl.ds(0, _K3)], sidx[b], sS[b]).wait()
        pltpu.make_async_copy(dst.at[pl.ds(0, _K3)], didx[b], sD[b]).wait()

    def issue_gather(b, j):
        @pl.when(jnp.int32(j) >= 2)
        def _():
            pltpu.make_async_copy(ra[b], acc.at[dS[b]], sW[b]).wait()
        pltpu.async_copy(ta.at[sidx[b]], ra[b], sG[b])
        pltpu.async_copy(tb.at[didx[b]], rb[b], sE[b])
        pltpu.async_copy(ch.at[pl.ds(base(j), _K3)], cc[b], sC[b])

    def wait_gather(b):
        pltpu.make_async_copy(ta.at[sidx[b]], ra[b], sG[b]).wait()
        pltpu.make_async_copy(tb.at[didx[b]], rb[b], sE[b]).wait()
        pltpu.make_async_copy(ch.at[pl.ds(0, _K3)], cc[b], sC[b]).wait()

    def consume(b, j):
        def efn(i, c2):
            for k2 in range(2):
                k = i * 2 + k2
                for hh in (0, 16):
                    v = (ra[b][k, pl.ds(hh, 16)] + rb[b][k, pl.ds(hh, 16)]
                         + cc[b][k, pl.ds(hh, 16)])
                    ra[b][k, pl.ds(hh, 16)] = jnp.maximum(v, 0.0)
            return c2
        lax.fori_loop(0, _K3 // 2, efn, 0)

        def cidx(i, c2):
            dS[b][pl.ds(i * 16, 16)] = didx[b][pl.ds(i * 16, 16)]
            return c2
        lax.fori_loop(0, _K3 // 16, cidx, 0)
        pltpu.async_copy(ra[b], acc.at[dS[b]], sW[b], add=True)

    _pipelined(T, issue_idx, wait_idx, issue_gather, wait_gather, consume)
    pltpu.make_async_copy(ra[0], acc.at[dS[0]], sW[0]).wait()
    pltpu.make_async_copy(ra[1], acc.at[dS[1]], sW[1]).wait()
    plsc.subcore_barrier()
    pltpu.sync_copy(acc.at[pl.ds(s * NPC, NPC)],
                    out.at[c, pl.ds(s * NPC, NPC)])


@functools.lru_cache(maxsize=None)
def _sc_escatter():
    return pl.kernel(
        _sc_escatter_body,
        out_type=jax.ShapeDtypeStruct((NC2, NACC, H), _f32),
        mesh=_get_mesh(),
        compiler_params=pltpu.CompilerParams(use_tc_tiling_on_sc=False),
        scratch_types=(
            [pltpu.VMEM((_K3,), jnp.int32)] * 6
            + [pltpu.VMEM((_K3, H), _f32)] * 6
            + [pltpu.VMEM_SHARED((NACC, H), _f32)]
            + [pltpu.SemaphoreType.DMA] * 12
        ),
    )


def _rows(d, bn=BN):
    return pl.BlockSpec((bn, d), lambda i: (i, 0))


def _full(a):
    nd = a.ndim
    return pl.BlockSpec(a.shape, lambda i, _nd=nd: (0,) * _nd)


def _agg_spec(bn=BN):
    return pl.BlockSpec((NC2, bn, H), lambda i: (0, i, 0))


def _relu(x):
    return jnp.maximum(x, 0.0)


def _sigm(x):
    return 1.0 / (1.0 + jnp.exp(-x))


def _tc(body, in_specs, out_shapes, out_specs, grid):
    return pl.pallas_call(
        body,
        grid=(grid,),
        in_specs=in_specs,
        out_specs=out_specs,
        out_shape=out_shapes,
    )


def _sds(*shape):
    return jax.ShapeDtypeStruct(shape, _f32)


def kernel(discrete_x, continous_x, edge_index, edge_attr, churn_date, t, params):
    p = params
    src = edge_index[0]
    dst = edge_index[1]
    cd = churn_date[:, 0]
    cdp = cd * p['g1_wg'] + 0.5 * p['g1_bg']
    cdm = cd * p['g1_wg'] - 0.5 * p['g1_bg']

    counts2, gate = _sc_deg_gate()(src, dst, cdp, cdm)
    counts2 = counts2[:, :N, None]

    elW_A = p['el_W'][:H]
    elW_B = p['el_W'][H:2 * H]
    elW_C = p['el_W'][2 * H:]

    def tc1(disc, cont, cnt2, Wd, bd, Wc, bc, g0W, g0b, g1W,
            xd_o, xc_o, h1_o, hn1_o, norm_o, invdeg_o):
        cnt = cnt2[0] + cnt2[1]
        norm = lax.rsqrt(cnt + 1.0)
        invdeg = 1.0 / jnp.maximum(cnt, 1.0)
        norm_o[...] = norm
        invdeg_o[...] = invdeg
        xd = jnp.dot(disc[...], Wd[...], preferred_element_type=_f32) + bd[...]
        xd_o[...] = xd
        cont_v = cont[...]
        xcs = [_relu(jnp.dot(cont_v[:, 16 * i:16 * (i + 1)], Wc[...],
                             preferred_element_type=_f32) + bc[...])
               for i in range(3)]
        xc = jnp.concatenate(xcs, axis=-1)
        xc_o[...] = xc
        xg = _relu(jnp.dot(jnp.concatenate([xd, xc], axis=-1), g0W[...],
                           preferred_element_type=_f32) + g0b[...])
        h1 = jnp.dot(xg, g1W[...], preferred_element_type=_f32)
        h1_o[...] = h1
        hn1_o[...] = h1 * norm

    xd, xc, h1, hn1, normv, invdeg = _tc(
        tc1,
        [_rows(16), _rows(48), pl.BlockSpec((NC2, BN, 1), lambda i: (0, i, 0)),
         _full(p['W_d']), _full(p['b_d']), _full(p['W_c']), _full(p['b_c']),
         _full(p['g0_W']), _full(p['g0_b']), _full(p['gcn1_W'])],
        (_sds(N, 10), _sds(N, 24), _sds(N, H), _sds(N, H), _sds(N, 1), _sds(N, 1)),
        (_rows(10), _rows(24), _rows(H), _rows(H), _rows(1), _rows(1)),
        25,
    )(discrete_x, continous_x, counts2, p['W_d'], p['b_d'], p['W_c'], p['b_c'],
      p['g0_W'], p['g0_b'], p['gcn1_W'])

    agg1 = _sc_scatter_plain(src, dst, hn1)

    def tc2(agg, h1r, nr, g2W, b1, xg0_o, h2_o, hn2_o):
        nv = nr[...]
        xg0 = _relu((agg[0] + agg[1]) * nv + h1r[...] * nv * nv + b1[...])
        xg0_o[...] = xg0
        h2 = jnp.dot(xg0, g2W[...], preferred_element_type=_f32)
        h2_o[...] = h2
        hn2_o[...] = h2 * nv

    xg0, h2, hn2 = _tc(
        tc2,
        [_agg_spec(), _rows(H), _rows(1), _full(p['gcn2_W']), _full(p['gcn1_b'])],
        (_sds(N, H), _sds(N, H), _sds(N, H)),
        (_rows(H), _rows(H), _rows(H)),
        25,
    )(agg1, h1, normv, p['gcn2_W'], p['gcn1_b'])

    agg2 = _sc_scatter_plain(src, dst, hn2)

    def tc3(agg, h2r, nr, xdr, xcr, xg0r, b2,
            res1W, res1b, res2W, res2b, cr1w, cr1b, cr2w, cr2b, fuW, fub,
            si0W, si0b, eWA, eWB, el1Wx, el1Wr, el1b, c0W, c0b, g1Wm, g1Wr, g1b,
            hci_o, xsi_o, A_o, B_o, hx1_o, xr1b_o, xns_o, hm1_o, xrn1b_o):
        nv = nr[...]
        xg1 = _relu((agg[0] + agg[1]) * nv + h2r[...] * nv * nv + b2[...])
        x = jnp.concatenate([xdr[...], xcr[...], xg0r[...] + xg1], axis=-1)
        h1r = _relu(jnp.dot(x, res1W[...], preferred_element_type=_f32) + res1b[...]) + x
        x_deep = _relu(jnp.dot(h1r, res2W[...], preferred_element_type=_f32) + res2b[...]) + h1r
        xl = x
        s1 = jnp.dot(xl, cr1w[...], preferred_element_type=_f32)
        xl = x * s1 + cr1b[...] + xl
        s2 = jnp.dot(xl, cr2w[...], preferred_element_type=_f32)
        xl = x * s2 + cr2b[...] + xl
        hci_o[...] = _relu(jnp.dot(x_deep + xl, fuW[...], preferred_element_type=_f32) + fub[...])
        xsi = _relu(jnp.dot(x, si0W[...], preferred_element_type=_f32) + si0b[...])
        xsi_o[...] = xsi
        A_o[...] = jnp.dot(xsi, eWA[...], preferred_element_type=_f32)
        B_o[...] = jnp.dot(xsi, eWB[...], preferred_element_type=_f32)
        hx1_o[...] = jnp.dot(xsi, el1Wx[...], preferred_element_type=_f32)
        xr1b_o[...] = jnp.dot(xsi, el1Wr[...], preferred_element_type=_f32) + el1b[...]
        xns = _relu(jnp.dot(x, c0W[...], preferred_element_type=_f32) + c0b[...])
        xns_o[...] = xns
        hm1_o[...] = jnp.dot(xns, g1Wm[...], preferred_element_type=_f32)
        xrn1b_o[...] = jnp.dot(xns, g1Wr[...], preferred_element_type=_f32) + g1b[...]

    cr1w = p['cr1_w'][:, None]
    cr2w = p['cr2_w'][:, None]
    h_ci, x_si, A, B, hx1, xr1b, x_ns, hm1, xrn1b = _tc(
        tc3,
        [_agg_spec(), _rows(H), _rows(1), _rows(10), _rows(24), _rows(H),
         _full(p['gcn2_b']),
         _full(p['res1_W']), _full(p['res1_b']), _full(p['res2_W']), _full(p['res2_b']),
         _full(cr1w), _full(p['cr1_b']), _full(cr2w), _full(p['cr2_b']),
         _full(p['fu_W']), _full(p['fu_b']),
         _full(p['si0_W']), _full(p['si0_b']), _full(elW_A), _full(elW_B),
         _full(p['el1_Wx']), _full(p['el1_Wr']), _full(p['el1_b']),
         _full(p['c0_W']), _full(p['c0_b']),
         _full(p['g1_Wm']), _full(p['g1_Wr']), _full(p['g1_b'])],
        tuple(_sds(N, H) for _ in range(9)),
        tuple(_rows(H) for _ in range(9)),
        25,
    )(agg2, h2, normv, xd, xc, xg0, p['gcn2_b'],
      p['res1_W'], p['res1_b'], p['res2_W'], p['res2_b'],
      cr1w, p['cr1_b'], cr2w, p['cr2_b'], p['fu_W'], p['fu_b'],
      p['si0_W'], p['si0_b'], elW_A, elW_B,
      p['el1_Wx'], p['el1_Wr'], p['el1_b'], p['c0_W'], p['c0_b'],
      p['g1_Wm'], p['g1_Wr'], p['g1_b'])

    def tce1(ea, W, b, C_o):
        C_o[...] = jnp.dot(ea[...], W[...], preferred_element_type=_f32) + b[...]

    C = _tc(tce1, [_rows(16, BE), _full(elW_C), _full(p['el_b'])],
            _sds(E, H), _rows(H, BE), 100)(edge_attr, elW_C, p['el_b'])

    S_e = _sc_escatter()(src, dst, A, B, C)

    sagg1 = _sc_scatter_plain(src, dst, hx1)

    def tc4(agg, se, idg, xr1br, We1, el2Wx, el2Wr, el2b, xsi0_o, hx2_o, xr2b_o):
        eterm = jnp.dot(se[0] + se[1], We1[...], preferred_element_type=_f32)
        xsi0 = _relu((agg[0] + agg[1] + eterm) * idg[...] + xr1br[...])
        xsi0_o[...] = xsi0
        hx2_o[...] = jnp.dot(xsi0, el2Wx[...], preferred_element_type=_f32)
        xr2b_o[...] = jnp.dot(xsi0, el2Wr[...], preferred_element_type=_f32) + el2b[...]

    x_si0, hx2, xr2b = _tc(
        tc4,
        [_agg_spec(), _agg_spec(), _rows(1), _rows(H), _full(p['el1_We']),
         _full(p['el2_Wx']), _full(p['el2_Wr']), _full(p['el2_b'])],
        (_sds(N, H), _sds(N, H), _sds(N, H)), (_rows(H), _rows(H), _rows(H)), 25,
    )(sagg1, S_e, invdeg, xr1b, p['el1_We'],
      p['el2_Wx'], p['el2_Wr'], p['el2_b'])

    sagg2 = _sc_scatter_plain(src, dst, hx2)

    def tc5(agg, se, idg, xr2br, xsi0r, We2, TW, Tb, hsi_o, predT_o):
        eterm = jnp.dot(se[0] + se[1], We2[...], preferred_element_type=_f32)
        xsi1 = _relu((agg[0] + agg[1] + eterm) * idg[...] + xr2br[...])
        hsi = xsi0r[...] + xsi1
        hsi_o[...] = hsi
        predT_o[...] = _sigm(jnp.dot(hsi, TW[...], preferred_element_type=_f32) + Tb[...])

    h_si, pred_T = _tc(
        tc5,
        [_agg_spec(), _agg_spec(), _rows(1), _rows(H), _rows(H),
         _full(p['el2_We']), _full(p['T_W']), _full(p['T_b'])],
        (_sds(N, H), _sds(N, 1)), (_rows(H), _rows(1)), 25,
    )(sagg2, S_e, invdeg, xr2b, x_si0, p['el2_We'], p['T_W'], p['T_b'])

    gagg1 = _sc_scatter_gate(src, dst, hm1, gate)

    def tc6(agg, idg, xrn1br, g1Wm, g1Wr, g1b, xns0_o, hm2_o, xrn2b_o):
        xns0 = _relu((agg[0] + agg[1]) * idg[...] + xrn1br[...])
        xns0_o[...] = xns0
        hm2_o[...] = jnp.dot(xns0, g1Wm[...], preferred_element_type=_f32)
        xrn2b_o[...] = jnp.dot(xns0, g1Wr[...], preferred_element_type=_f32) + g1b[...]

    x_ns0, hm2, xrn2b = _tc(
        tc6,
        [_agg_spec(), _rows(1), _rows(H),
         _full(p['g1_Wm']), _full(p['g1_Wr']), _full(p['g1_b'])],
        (_sds(N, H), _sds(N, H), _sds(N, H)), (_rows(H), _rows(H), _rows(H)), 25,
    )(gagg1, invdeg, xrn1b, p['g1_Wm'], p['g1_Wr'], p['g1_b'])

    gagg2 = _sc_scatter_gate(src, dst, hm2, gate)

    def tc7(agg, idg, xrn2br, xns0r, hcir, hsir, tr,
            a0W, a0b, a1W, a1b, y0hW, y0hb, y0oW, y0ob, y1hW, y1hb, y1oW, y1ob,
            py_o, pycf_o, py0_o, py1_o):
        xns1 = _relu((agg[0] + agg[1]) * idg[...] + xrn2br[...])
        hns = xns0r[...] + xns1
        hci = hcir[...]
        hsi = hsir[...]
        h = jnp.concatenate([hci, hsi, hns], axis=-1)
        a0 = jax.nn.softmax(jnp.dot(h, a0W[...], preferred_element_type=_f32) + a0b[...], axis=-1)
        py0 = a0[:, :H] * hci + a0[:, H:2 * H] * hsi + a0[:, 2 * H:] * hns
        a1 = jax.nn.softmax(jnp.dot(h, a1W[...], preferred_element_type=_f32) + a1b[...], axis=-1)
        py1 = a1[:, :H] * hci + a1[:, H:2 * H] * hsi + a1[:, 2 * H:] * hns
        py0 = _sigm(jnp.dot(_relu(jnp.dot(py0, y0hW[...], preferred_element_type=_f32) + y0hb[...]),
                            y0oW[...], preferred_element_type=_f32) + y0ob[...])
        py1 = _sigm(jnp.dot(_relu(jnp.dot(py1, y1hW[...], preferred_element_type=_f32) + y1hb[...]),
                            y1oW[...], preferred_element_type=_f32) + y1ob[...])
        tv = tr[...]
        py_o[...] = (1.0 - tv) * py0 + tv * py1
        pycf_o[...] = tv * py0 + (1.0 - tv) * py1
        py0_o[...] = py0
        py1_o[...] = py1

    pred_y, pred_y_cf, pred_y0, pred_y1 = _tc(
        tc7,
        [_agg_spec(), _rows(1), _rows(H), _rows(H), _rows(H), _rows(H), _rows(1),
         _full(p['a0_W']), _full(p['a0_b']), _full(p['a1_W']), _full(p['a1_b']),
         _full(p['y0h_W']), _full(p['y0h_b']), _full(p['y0o_W']), _full(p['y0o_b']),
         _full(p['y1h_W']), _full(p['y1h_b']), _full(p['y1o_W']), _full(p['y1o_b'])],
        (_sds(N, 1), _sds(N, 1), _sds(N, 1), _sds(N, 1)),
        (_rows(1), _rows(1), _rows(1), _rows(1)),
        25,
    )(gagg2, invdeg, xrn2b, x_ns0, h_ci, h_si, t,
      p['a0_W'], p['a0_b'], p['a1_W'], p['a1_b'],
      p['y0h_W'], p['y0h_b'], p['y0o_W'], p['y0o_b'],
      p['y1h_W'], p['y1h_b'], p['y1o_W'], p['y1o_b'])

    return (pred_y, pred_y_cf, pred_y0, pred_y1, pred_T, h_ci, h_si)
